# Initial kernel scaffold; baseline (speedup 1.0000x reference)
import functools

import jax
import jax.numpy as jnp
from jax import lax
from jax.experimental import pallas as pl
from jax.experimental.pallas import tpu as pltpu
from jax.experimental.pallas import tpu_sc as plsc

SN = 10000
EN = 10000
DIM = 128
E = 320000
NB = 4096
TEMP = 0.2
CONTROL = 0.3
NT = 16
EPT = E // NT
K1 = 160
NCH1 = EPT // K1
K2 = 200
NCH2 = EPT // K2
ROWS_PT = SN // NT
K0 = 2000
NCH0 = (E // 8) // K0
SEL_PT = NB // NT

_mesh = plsc.VectorSubcoreMesh(core_axis_name="c", subcore_axis_name="s")


def _f32(shape):
    return jax.ShapeDtypeStruct(shape, jnp.float32)


def _takev(v, idx):
    return v.at[idx].get(mode="promise_in_bounds")


def _hsum16(accs):
    lane = lax.iota(jnp.int32, 16)
    cur = accs
    for fold, rotk in ((8, 8), (4, 12), (2, 14), (1, 15)):
        fidx = jnp.bitwise_and(lane + fold, 15)
        ridx = jnp.bitwise_and(lane + rotk, 15)
        m = jnp.bitwise_and(lane, 2 * fold - 1) < fold
        nxt = []
        for a, b in zip(cur[0::2], cur[1::2]):
            fa = a + _takev(a, fidx)
            fb = b + _takev(b, fidx)
            nxt.append(jnp.where(m, fa, _takev(fb, ridx)))
        cur = nxt
    perm = (jnp.bitwise_and(lane, 1) * 8 + jnp.bitwise_and(lane, 2) * 2
            + lax.shift_right_logical(jnp.bitwise_and(lane, 4), 1)
            + lax.shift_right_logical(jnp.bitwise_and(lane, 8), 3))
    return _takev(cur[0], perm)


def _p0_body(u1, i1, u0, i0, zeros4, ones_h, degp,
             idxb, onesb, a0, a1, a2, a3):
    cid = lax.axis_index("c")
    sid = lax.axis_index("s")
    accs = (a0, a1, a2, a3)
    arrays = (u1, i1, u0, i0)

    @pl.when(sid == 0)
    def _():
        for k in range(4):
            pltpu.sync_copy(zeros4.at[k], accs[k])

    pltpu.sync_copy(ones_h, onesb)
    plsc.subcore_barrier()

    a = sid // 4
    q = jnp.remainder(sid, 4)
    wrk = cid * 4 + q

    def chunk(c, carry):
        base = wrk * (E // 8) + c * K0
        for k in range(4):
            @pl.when(a == k)
            def _(k=k):
                pltpu.sync_copy(arrays[k].at[pl.ds(base, K0)], idxb)
                pltpu.sync_copy(onesb, accs[k].at[idxb], add=True)
        return carry

    lax.fori_loop(0, NCH0, chunk, 0)
    plsc.subcore_barrier()
    for k in range(4):
        @pl.when(sid == k)
        def _(k=k):
            pltpu.sync_copy(accs[k], degp.at[cid, k])


def _pass0(u1, i1, u0, i0):
    zeros4 = jnp.zeros((4, SN), jnp.float32)
    ones_h = jnp.ones((K0,), jnp.float32)
    return pl.kernel(
        _p0_body,
        out_type=_f32((2, 4, SN)),
        mesh=_mesh,
        scratch_types=[
            pltpu.VMEM((K0,), jnp.int32),
            pltpu.VMEM((K0,), jnp.float32),
            pltpu.VMEM_SHARED((SN,), jnp.float32),
            pltpu.VMEM_SHARED((SN,), jnp.float32),
            pltpu.VMEM_SHARED((SN,), jnp.float32),
            pltpu.VMEM_SHARED((SN,), jnp.float32),
        ],
    )(u1, i1, u0, i0, zeros4, ones_h)


def _gate_chunk(base, hbm, bufs, w2v):
    (d_h, s_h, lg_h, dstT, srcT, su_t, si_t, v_h, acc) = hbm
    (idx_d, idx_s, lg_e, su_e, si_e, dst_rows, src_rows, msg, vout,
     s1, s2, s3, s4) = bufs
    pltpu.sync_copy(d_h.at[pl.ds(base, K1)], idx_d)
    pltpu.sync_copy(s_h.at[pl.ds(base, K1)], idx_s)
    pltpu.sync_copy(lg_h.at[pl.ds(base, K1)], lg_e)
    c1 = pltpu.async_copy(dstT.at[idx_d], dst_rows, s1)
    c2 = pltpu.async_copy(srcT.at[idx_s], src_rows, s2)
    c3 = pltpu.async_copy(su_t.at[idx_d], su_e, s3)
    c4 = pltpu.async_copy(si_t.at[idx_s], si_e, s4)
    c1.wait()
    c2.wait()
    c3.wait()
    c4.wait()

    def group(g, carry):
        e0 = g * 16
        accs = []
        for j in range(16):
            e = e0 + j
            acc16 = None
            for f in range(8):
                av = dst_rows[e, pl.ds(16 * f, 16)]
                bv = src_rows[e, pl.ds(16 * f, 16)]
                h = jnp.maximum(av + bv, 0.0)
                t = h * w2v[f]
                acc16 = t if acc16 is None else acc16 + t
            accs.append(acc16)
        logit = _hsum16(accs)
        gv = logit + lg_e[pl.ds(e0, 16)]
        w = CONTROL / (1.0 + jnp.exp(-gv)) + (1.0 - CONTROL)
        v = su_e[pl.ds(e0, 16)] * si_e[pl.ds(e0, 16)] * w
        vout[pl.ds(e0, 16)] = v
        for j in range(16):
            e = e0 + j
            vs = jnp.full((16,), vout[e], jnp.float32)
            for f in range(8):
                msg[e, pl.ds(16 * f, 16)] = vs * src_rows[e, pl.ds(128 + 16 * f, 16)]
        return carry

    lax.fori_loop(0, K1 // 16, group, 0)
    pltpu.sync_copy(vout, v_h.at[pl.ds(base, K1)])
    pltpu.sync_copy(msg, acc.at[idx_d], add=True)


def _p1_side(sid, polars, init_tab, out_tab, acc, bufs, w2buf, w2s, w2row):
    r0 = sid * ROWS_PT
    pltpu.sync_copy(init_tab.at[pl.ds(r0, ROWS_PT)], acc.at[pl.ds(r0, ROWS_PT)])
    plsc.subcore_barrier()
    for p, hbm in enumerate(polars):
        pltpu.sync_copy(w2s.at[w2row[p]], w2buf)
        w2v = [w2buf[pl.ds(16 * f, 16)] for f in range(8)]

        def chunk(c, carry, hbm=hbm, w2v=w2v):
            base = sid * EPT + c * K1
            _gate_chunk(base, hbm + (acc,), bufs, w2v)
            return carry

        lax.fori_loop(0, NCH1, chunk, 0)
    plsc.subcore_barrier()
    pltpu.sync_copy(acc.at[pl.ds(r0, ROWS_PT)], out_tab.at[pl.ds(r0, ROWS_PT)])


def _p1_body(u1, i1, u0, i0, lg_ui1, lg_iu1, lg_ui0, lg_iu0,
             d1u, s1i, d1i, s1u, d0u, s0i, d0i, s0u,
             su1, si1, su0, si0, w2s, sW, eW,
             ue1, ie1, v_ui1, v_ui0, v_iu1, v_iu0,
             idx_d, idx_s, lg_e, su_e, si_e, dst_rows, src_rows, msg, vout,
             w2buf, acc, s1_, s2_, s3_, s4_):
    cid = lax.axis_index("c")
    sid = lax.axis_index("s")
    bufs = (idx_d, idx_s, lg_e, su_e, si_e, dst_rows, src_rows, msg, vout,
            s1_, s2_, s3_, s4_)

    @pl.when(cid == 0)
    def _():
        polars = (
            (u1, i1, lg_ui1, d1u, s1i, su1, si1, v_ui1),
            (u0, i0, lg_ui0, d0u, s0i, su0, si0, v_ui0),
        )
        _p1_side(sid, polars, sW, ue1, acc, bufs, w2buf, w2s, (0, 1))

    @pl.when(cid == 1)
    def _():
        polars = (
            (i1, u1, lg_iu1, d1i, s1u, si1, su1, v_iu1),
            (i0, u0, lg_iu0, d0i, s0u, si0, su0, v_iu0),
        )
        _p1_side(sid, polars, eW, ie1, acc, bufs, w2buf, w2s, (0, 1))


def _pass1(u1, i1, u0, i0, lgs, tabs, sus, w2s, sW, eW):
    return pl.kernel(
        _p1_body,
        out_type=[_f32((SN, DIM)), _f32((EN, DIM)),
                  _f32((E,)), _f32((E,)), _f32((E,)), _f32((E,))],
        mesh=_mesh,
        scratch_types=[
            pltpu.VMEM((K1,), jnp.int32),
            pltpu.VMEM((K1,), jnp.int32),
            pltpu.VMEM((K1,), jnp.float32),
            pltpu.VMEM((K1,), jnp.float32),
            pltpu.VMEM((K1,), jnp.float32),
            pltpu.VMEM((K1, DIM), jnp.float32),
            pltpu.VMEM((K1, 2 * DIM), jnp.float32),
            pltpu.VMEM((K1, DIM), jnp.float32),
            pltpu.VMEM((K1,), jnp.float32),
            pltpu.VMEM((DIM,), jnp.float32),
            pltpu.VMEM_SHARED((SN, DIM), jnp.float32),
            pltpu.SemaphoreType.DMA,
            pltpu.SemaphoreType.DMA,
            pltpu.SemaphoreType.DMA,
            pltpu.SemaphoreType.DMA,
        ],
    )(u1, i1, u0, i0, *lgs, *tabs, *sus, w2s, sW, eW)


def _p2_side(sid, polars, init_tab, gat_tab, sel_idx, sel_out, acc, bufs):
    (idx_d, idx_s, vbuf, src_rows, msg, selidx, selrows, s1, s2) = bufs
    r0 = sid * ROWS_PT
    pltpu.sync_copy(init_tab.at[pl.ds(r0, ROWS_PT)], acc.at[pl.ds(r0, ROWS_PT)])
    plsc.subcore_barrier()
    for d_h, s_h, v_h in polars:
        def chunk(c, carry, d_h=d_h, s_h=s_h, v_h=v_h):
            base = sid * EPT + c * K2
            pltpu.sync_copy(d_h.at[pl.ds(base, K2)], idx_d)
            pltpu.sync_copy(s_h.at[pl.ds(base, K2)], idx_s)
            pltpu.sync_copy(v_h.at[pl.ds(base, K2)], vbuf)
            pltpu.async_copy(gat_tab.at[idx_s], src_rows, s1).wait()

            def edge(e, cc):
                vs = jnp.full((16,), vbuf[e], jnp.float32)
                for f in range(8):
                    msg[e, pl.ds(16 * f, 16)] = vs * src_rows[e, pl.ds(16 * f, 16)]
                return cc

            lax.fori_loop(0, K2, edge, 0)
            pltpu.sync_copy(msg, acc.at[idx_d], add=True)
            return carry

        lax.fori_loop(0, NCH2, chunk, 0)
    plsc.subcore_barrier()
    h0 = sid * SEL_PT
    pltpu.sync_copy(sel_idx.at[pl.ds(h0, SEL_PT)], selidx)
    pltpu.async_copy(acc.at[selidx], selrows, s2).wait()
    pltpu.sync_copy(selrows, sel_out.at[pl.ds(h0, SEL_PT)])


def _p2_body(u1, i1, u0, i0, v_ui1, v_ui0, v_iu1, v_iu0,
             ue1, ie1, stu_id, exer_id, disc_t,
             stu_e, ex_e, disc_e,
             idx_d, idx_s, vbuf, src_rows, msg, selidx, selrows, drows,
             acc, s1, s2):
    cid = lax.axis_index("c")
    sid = lax.axis_index("s")
    bufs = (idx_d, idx_s, vbuf, src_rows, msg, selidx, selrows, s1, s2)

    @pl.when(cid == 0)
    def _():
        polars = ((u1, i1, v_ui1), (u0, i0, v_ui0))
        _p2_side(sid, polars, ue1, ie1, stu_id, stu_e, acc, bufs)

    @pl.when(cid == 1)
    def _():
        polars = ((i1, u1, v_iu1), (i0, u0, v_iu0))
        _p2_side(sid, polars, ie1, ue1, exer_id, ex_e, acc, bufs)
        h0 = sid * SEL_PT
        pltpu.async_copy(disc_t.at[selidx], drows, s1).wait()
        pltpu.sync_copy(drows, disc_e.at[pl.ds(h0, SEL_PT)])


def _pass2(u1, i1, u0, i0, vs, ue1, ie1, stu_id, exer_id, disc_t):
    return pl.kernel(
        _p2_body,
        out_type=[_f32((NB, DIM)), _f32((NB, DIM)), _f32((NB,))],
        mesh=_mesh,
        scratch_types=[
            pltpu.VMEM((K2,), jnp.int32),
            pltpu.VMEM((K2,), jnp.int32),
            pltpu.VMEM((K2,), jnp.float32),
            pltpu.VMEM((K2, DIM), jnp.float32),
            pltpu.VMEM((K2, DIM), jnp.float32),
            pltpu.VMEM((SEL_PT,), jnp.int32),
            pltpu.VMEM((SEL_PT, DIM), jnp.float32),
            pltpu.VMEM((SEL_PT,), jnp.float32),
            pltpu.VMEM_SHARED((SN, DIM), jnp.float32),
            pltpu.SemaphoreType.DMA,
            pltpu.SemaphoreType.DMA,
        ],
    )(u1, i1, u0, i0, *vs, ue1, ie1, stu_id, exer_id, disc_t)


def _dotT(x, w):
    return lax.dot_general(x, w, (((1,), (1,)), ((), ())),
                           preferred_element_type=jnp.float32)


def _prep_tables_body(sw, ew, w11, b11, w01, b01,
                      d1u, s1i, d1i, s1u, d0u, s0i, d0i, s0u):
    swv = sw[...]
    ewv = ew[...]
    for wref, bref, du, si_, di, su_ in (
            (w11, b11, d1u, s1i, d1i, s1u), (w01, b01, d0u, s0i, d0i, s0u)):
        w = wref[...]
        a = _dotT(swv, w[:, :DIM]) + bref[...]
        b = _dotT(ewv, w[:, DIM:])
        du[...] = a
        di[...] = b
        si_[:, :DIM] = b
        si_[:, DIM:] = ewv
        su_[:, :DIM] = a
        su_[:, DIM:] = swv


def _prep_tables(sW, eW, l1_W1, l1_b1, l0_W1, l0_b1):
    R = 1000
    grid = SN // R
    blk = pl.BlockSpec((R, DIM), lambda i: (i, 0))
    blk2 = pl.BlockSpec((R, 2 * DIM), lambda i: (i, 0))
    full = lambda s: pl.BlockSpec(s, lambda i: tuple(0 for _ in s))
    return pl.pallas_call(
        _prep_tables_body,
        grid=(grid,),
        in_specs=[blk, blk, full((DIM, 2 * DIM)), full((1, DIM)),
                  full((DIM, 2 * DIM)), full((1, DIM))],
        out_specs=[blk, blk2, blk, blk2, blk, blk2, blk, blk2],
        out_shape=[_f32((SN, DIM)), _f32((EN, 2 * DIM)),
                   _f32((EN, DIM)), _f32((SN, 2 * DIM)),
                   _f32((SN, DIM)), _f32((EN, 2 * DIM)),
                   _f32((EN, DIM)), _f32((SN, 2 * DIM))],
    )(sW, eW, l1_W1, l1_b1.reshape(1, DIM), l0_W1, l0_b1.reshape(1, DIM))


def _prep_small_body(degp, edisc, su4, disc):
    d = degp[...]
    su4[...] = lax.rsqrt(d[0] + d[1] + 1.0)
    disc[...] = jax.nn.sigmoid(edisc[...].reshape(1, EN)) * 10.0


def _prep_small(degp, e_disc_W):
    return pl.pallas_call(
        _prep_small_body,
        out_shape=[_f32((4, SN)), _f32((1, EN))],
    )(degp, e_disc_W)


def _prep_edge_body(e1, e2, e3, e4, b21, b20, o1, o2, o3, o4):
    def lg(x, b2):
        xc = jnp.clip(x, 1e-6, 1.0 - 1e-6)
        return 5.0 * (jnp.log(xc) - jnp.log1p(-xc) + b2)
    b1v = b21[0, 0]
    b0v = b20[0, 0]
    o1[...] = lg(e1[...], b1v)
    o2[...] = lg(e2[...], b1v)
    o3[...] = lg(e3[...], b0v)
    o4[...] = lg(e4[...], b0v)


def _prep_edge(eps_ui_1, eps_iu_1, eps_ui_0, eps_iu_0, l1_b2, l0_b2):
    R = 250
    W = 128
    grid = E // (R * W)
    blk = pl.BlockSpec((R, W), lambda i: (i, 0))
    full = pl.BlockSpec((1, 1), lambda i: (0, 0))
    rs = lambda x: x.reshape(E // W, W)
    outs = pl.pallas_call(
        _prep_edge_body,
        grid=(grid,),
        in_specs=[blk, blk, blk, blk, full, full],
        out_specs=[blk, blk, blk, blk],
        out_shape=[_f32((E // W, W))] * 4,
    )(rs(eps_ui_1), rs(eps_iu_1), rs(eps_ui_0), rs(eps_iu_0),
      l1_b2.reshape(1, 1), l0_b2.reshape(1, 1))
    return [o.reshape(E) for o in outs]


def _head_body(stu, ex, disc, knr, knw, p1w, p1b, p2w, p2b, p3w, p3b, out):
    stat = jax.nn.sigmoid(_dotT(stu[...], knw[...]))
    kdiff = jax.nn.sigmoid(_dotT(ex[...], knw[...]))
    x = disc[...] * (stat - kdiff) * knr[...]
    h1 = jax.nn.sigmoid(_dotT(x, jnp.abs(p1w[...])) + p1b[...])
    h2 = jax.nn.sigmoid(_dotT(h1, jnp.abs(p2w[...])) + p2b[...])
    out[...] = jax.nn.sigmoid(_dotT(h2, jnp.abs(p3w[...])) + p3b[...])


def _head(stu_e, ex_e, disc_e, kn_r, knowledge_W,
          pn1_W, pn1_b, pn2_W, pn2_b, pn3_W, pn3_b):
    R = 512
    grid = NB // R
    blk = pl.BlockSpec((R, DIM), lambda i: (i, 0))
    blk1 = pl.BlockSpec((R, 1), lambda i: (i, 0))
    full = lambda s: pl.BlockSpec(s, lambda i: tuple(0 for _ in s))
    out = pl.pallas_call(
        _head_body,
        grid=(grid,),
        in_specs=[blk, blk, blk1, blk,
                  full((DIM, DIM)),
                  full((256, DIM)), full((1, 256)),
                  full((DIM, 256)), full((1, DIM)),
                  full((1, DIM)), full((1, 1))],
        out_specs=blk1,
        out_shape=_f32((NB, 1)),
    )(stu_e, ex_e, disc_e.reshape(NB, 1), kn_r, knowledge_W,
      pn1_W, pn1_b.reshape(1, 256), pn2_W, pn2_b.reshape(1, DIM),
      pn3_W, pn3_b.reshape(1, 1))
    return out.reshape(NB)


def kernel(stu_id, exer_id, kn_r, edge_index_1, edge_index_0,
           eps_ui_1, eps_iu_1, eps_ui_0, eps_iu_0,
           student_W, exercise_W, knowledge_W, e_disc_W,
           l1_W1, l1_b1, l1_W2, l1_b2,
           l0_W1, l0_b1, l0_W2, l0_b2,
           pn1_W, pn1_b, pn2_W, pn2_b, pn3_W, pn3_b):
    u1 = edge_index_1[0]
    i1 = edge_index_1[1]
    u0 = edge_index_0[0]
    i0 = edge_index_0[1]

    degp = _pass0(u1, i1, u0, i0)
    su4, disc = _prep_small(degp, e_disc_W)
    su1, si1, su0, si0 = su4[0], su4[1], su4[2], su4[3]
    disc_t = disc.reshape(EN)

    tabs = _prep_tables(student_W, exercise_W, l1_W1, l1_b1, l0_W1, l0_b1)
    lgs = _prep_edge(eps_ui_1, eps_iu_1, eps_ui_0, eps_iu_0, l1_b2, l0_b2)

    w2s = jnp.concatenate([l1_W2, l0_W2], axis=0) * 5.0

    ue1, ie1, v_ui1, v_ui0, v_iu1, v_iu0 = _pass1(
        u1, i1, u0, i0, lgs, tabs, (su1, si1, su0, si0), w2s,
        student_W, exercise_W)

    stu_e, ex_e, disc_e = _pass2(
        u1, i1, u0, i0, (v_ui1, v_ui0, v_iu1, v_iu0),
        ue1, ie1, stu_id, exer_id, disc_t)

    return _head(stu_e, ex_e, disc_e, kn_r, knowledge_W,
                 pn1_W, pn1_b, pn2_W, pn2_b, pn3_W, pn3_b)


# speedup vs baseline: 3.2939x; 3.2939x over previous
import functools

import jax
import jax.numpy as jnp
from jax import lax
from jax.experimental import pallas as pl
from jax.experimental.pallas import tpu as pltpu
from jax.experimental.pallas import tpu_sc as plsc

SN = 10000
EN = 10000
DIM = 128
E = 320000
NB = 4096
TEMP = 0.2
CONTROL = 0.3
NT = 16
EPT = E // NT
K1 = 80
NCH1 = EPT // K1
K2 = 80
NCH2 = EPT // K2
SELC = 64
ROWS_SPLIT = [(t * 632, 632) for t in range(15)] + [(9480, 520)]
K0 = 2000
NCH0 = (E // 8) // K0
SEL_PT = NB // NT

_mesh = plsc.VectorSubcoreMesh(core_axis_name="c", subcore_axis_name="s")


def _f32(shape):
    return jax.ShapeDtypeStruct(shape, jnp.float32)


def _takev(v, idx):
    return v.at[idx].get(mode="promise_in_bounds")


def _rows_io(sid, src, dst):
    for t, (b, s) in enumerate(ROWS_SPLIT):
        @pl.when(sid == t)
        def _(b=b, s=s):
            pltpu.sync_copy(src.at[pl.ds(b, s)], dst.at[pl.ds(b, s)])


def _hsum16(accs):
    lane = lax.iota(jnp.int32, 16)
    cur = accs
    for fold, rotk in ((8, 8), (4, 12), (2, 14), (1, 15)):
        fidx = jnp.bitwise_and(lane + fold, 15)
        ridx = jnp.bitwise_and(lane + rotk, 15)
        m = jnp.bitwise_and(lane, 2 * fold - 1) < fold
        nxt = []
        for a, b in zip(cur[0::2], cur[1::2]):
            fa = a + _takev(a, fidx)
            fb = b + _takev(b, fidx)
            nxt.append(jnp.where(m, fa, _takev(fb, ridx)))
        cur = nxt
    perm = (jnp.bitwise_and(lane, 1) * 8 + jnp.bitwise_and(lane, 2) * 2
            + lax.shift_right_logical(jnp.bitwise_and(lane, 4), 1)
            + lax.shift_right_logical(jnp.bitwise_and(lane, 8), 3))
    return _takev(cur[0], perm)


def _p0_body(u1, i1, u0, i0, ones_h, degp,
             idxb, onesb, zbuf, a0, a1, a2, a3):
    cid = lax.axis_index("c")
    sid = lax.axis_index("s")
    accs = (a0, a1, a2, a3)
    arrays = (u1, i1, u0, i0)

    @pl.when(sid == 0)
    def _():
        def zfill(i, c):
            zbuf[pl.ds(i * 16, 16)] = jnp.zeros((16,), jnp.float32)
            return c
        lax.fori_loop(0, SN // 16, zfill, 0)
        for k in range(4):
            pltpu.sync_copy(zbuf, accs[k])

    pltpu.sync_copy(ones_h, onesb)
    plsc.subcore_barrier()

    a = sid // 4
    q = jnp.remainder(sid, 4)
    wrk = cid * 4 + q

    def chunk(c, carry):
        base = wrk * (E // 8) + c * K0
        for k in range(4):
            @pl.when(a == k)
            def _(k=k):
                pltpu.sync_copy(arrays[k].at[pl.ds(base, K0)], idxb)
                pltpu.sync_copy(onesb, accs[k].at[idxb], add=True)
        return carry

    lax.fori_loop(0, NCH0, chunk, 0)
    plsc.subcore_barrier()
    for k in range(4):
        @pl.when(sid == k)
        def _(k=k):
            pltpu.sync_copy(accs[k], zbuf)
            pltpu.sync_copy(zbuf, degp.at[pl.ds((cid * 4 + k) * SN, SN)])


def _pass0(u1, i1, u0, i0):
    ones_h = jnp.ones((K0,), jnp.float32)
    return pl.kernel(
        _p0_body,
        out_type=_f32((8 * SN,)),
        mesh=_mesh,
        scratch_types=[
            pltpu.VMEM((K0,), jnp.int32),
            pltpu.VMEM((K0,), jnp.float32),
            pltpu.VMEM((SN,), jnp.float32),
            pltpu.VMEM_SHARED((SN,), jnp.float32),
            pltpu.VMEM_SHARED((SN,), jnp.float32),
            pltpu.VMEM_SHARED((SN,), jnp.float32),
            pltpu.VMEM_SHARED((SN,), jnp.float32),
        ],
    )(u1, i1, u0, i0, ones_h)


def _gate_chunk(base, hbm, bufs, w2v):
    (d_h, s_h, lg_h, dstT, srcT, su_t, si_t, v_h, acc) = hbm
    (idx_d, idx_s, lg_e, su_e, si_e, dst_rows, src_rows, msg, vout,
     s1, s2, s3, s4) = bufs
    pltpu.sync_copy(d_h.at[pl.ds(base, K1)], idx_d)
    pltpu.sync_copy(s_h.at[pl.ds(base, K1)], idx_s)
    pltpu.sync_copy(lg_h.at[pl.ds(base, K1)], lg_e)
    c1 = pltpu.async_copy(dstT.at[idx_d], dst_rows, s1)
    c2 = pltpu.async_copy(srcT.at[idx_s], src_rows, s2)
    c3 = pltpu.async_copy(su_t.at[idx_d], su_e, s3)
    c4 = pltpu.async_copy(si_t.at[idx_s], si_e, s4)
    c1.wait()
    c2.wait()
    c3.wait()
    c4.wait()

    def group(g, carry):
        e0 = g * 16
        accs = []
        for j in range(16):
            e = e0 + j
            acc16 = None
            for f in range(8):
                av = dst_rows[e, pl.ds(16 * f, 16)]
                bv = src_rows[e, pl.ds(16 * f, 16)]
                h = jnp.maximum(av + bv, 0.0)
                t = h * w2v[f]
                acc16 = t if acc16 is None else acc16 + t
            accs.append(acc16)
        logit = _hsum16(accs)
        gv = logit + lg_e[pl.ds(e0, 16)]
        w = CONTROL / (1.0 + jnp.exp(-gv)) + (1.0 - CONTROL)
        v = su_e[pl.ds(e0, 16)] * si_e[pl.ds(e0, 16)] * w
        vout[pl.ds(e0, 16)] = v
        for j in range(16):
            e = e0 + j
            vs = _takev(v, jnp.full((16,), j, jnp.int32))
            for f in range(8):
                msg[e, pl.ds(16 * f, 16)] = vs * src_rows[e, pl.ds(128 + 16 * f, 16)]
        return carry

    lax.fori_loop(0, K1 // 16, group, 0)
    pltpu.sync_copy(vout, v_h.at[pl.ds(base, K1)])
    pltpu.sync_copy(msg, acc.at[idx_d], add=True)


def _p1_side(sid, polars, init_tab, out_tab, acc, bufs, w2buf, w2s, w2row):
    _rows_io(sid, init_tab, acc)
    plsc.subcore_barrier()
    for p, hbm in enumerate(polars):
        pltpu.sync_copy(w2s.at[pl.ds(w2row[p], 8)], w2buf)
        w2v = [w2buf[0, pl.ds(16 * f, 16)] for f in range(8)]

        def chunk(c, carry, hbm=hbm, w2v=w2v):
            base = sid * EPT + c * K1
            _gate_chunk(base, hbm + (acc,), bufs, w2v)
            return carry

        lax.fori_loop(0, NCH1, chunk, 0)
    plsc.subcore_barrier()
    _rows_io(sid, acc, out_tab)


def _p1_body(u1, i1, u0, i0, lg_ui1, lg_iu1, lg_ui0, lg_iu0,
             d1u, s1i, d1i, s1u, d0u, s0i, d0i, s0u,
             su1, si1, su0, si0, w2s, sW, eW,
             ue1, ie1, v_ui1, v_ui0, v_iu1, v_iu0,
             idx_d, idx_s, lg_e, su_e, si_e, dst_rows, src_rows, msg, vout,
             w2buf, acc, s1_, s2_, s3_, s4_):
    cid = lax.axis_index("c")
    sid = lax.axis_index("s")
    bufs = (idx_d, idx_s, lg_e, su_e, si_e, dst_rows, src_rows, msg, vout,
            s1_, s2_, s3_, s4_)

    @pl.when(cid == 0)
    def _():
        polars = (
            (u1, i1, lg_ui1, d1u, s1i, su1, si1, v_ui1),
            (u0, i0, lg_ui0, d0u, s0i, su0, si0, v_ui0),
        )
        _p1_side(sid, polars, sW, ue1, acc, bufs, w2buf, w2s, (0, 8))

    @pl.when(cid == 1)
    def _():
        polars = (
            (i1, u1, lg_iu1, d1i, s1u, si1, su1, v_iu1),
            (i0, u0, lg_iu0, d0i, s0u, si0, su0, v_iu0),
        )
        _p1_side(sid, polars, eW, ie1, acc, bufs, w2buf, w2s, (0, 8))


def _pass1(u1, i1, u0, i0, lgs, tabs, sus, w2s, sW, eW):
    return pl.kernel(
        _p1_body,
        out_type=[_f32((SN, DIM)), _f32((EN, DIM)),
                  _f32((E,)), _f32((E,)), _f32((E,)), _f32((E,))],
        mesh=_mesh,
        scratch_types=[
            pltpu.VMEM((K1,), jnp.int32),
            pltpu.VMEM((K1,), jnp.int32),
            pltpu.VMEM((K1,), jnp.float32),
            pltpu.VMEM((K1,), jnp.float32),
            pltpu.VMEM((K1,), jnp.float32),
            pltpu.VMEM((K1, DIM), jnp.float32),
            pltpu.VMEM((K1, 2 * DIM), jnp.float32),
            pltpu.VMEM((K1, DIM), jnp.float32),
            pltpu.VMEM((K1,), jnp.float32),
            pltpu.VMEM((8, DIM), jnp.float32),
            pltpu.VMEM_SHARED((SN, DIM), jnp.float32),
            pltpu.SemaphoreType.DMA,
            pltpu.SemaphoreType.DMA,
            pltpu.SemaphoreType.DMA,
            pltpu.SemaphoreType.DMA,
        ],
    )(u1, i1, u0, i0, *lgs, *tabs, *sus, w2s, sW, eW)


def _p2_side(sid, polars, init_tab, gat_tab, sel_idx, sel_out, acc, bufs,
             extra=None):
    (idx_d, idx_s, vbuf, src_rows, msg, selidx, selrows, drows, s1, s2) = bufs
    _rows_io(sid, init_tab, acc)
    plsc.subcore_barrier()
    for d_h, s_h, v_h in polars:
        def chunk(c, carry, d_h=d_h, s_h=s_h, v_h=v_h):
            base = sid * EPT + c * K2
            pltpu.sync_copy(d_h.at[pl.ds(base, K2)], idx_d)
            pltpu.sync_copy(s_h.at[pl.ds(base, K2)], idx_s)
            pltpu.sync_copy(v_h.at[pl.ds(base, K2)], vbuf)
            pltpu.async_copy(gat_tab.at[idx_s], src_rows, s1).wait()

            def group(g, cc):
                e0 = g * 16
                v16 = vbuf[pl.ds(e0, 16)]
                for j in range(16):
                    e = e0 + j
                    vs = _takev(v16, jnp.full((16,), j, jnp.int32))
                    for f in range(8):
                        msg[e, pl.ds(16 * f, 16)] = vs * src_rows[e, pl.ds(16 * f, 16)]
                return cc

            lax.fori_loop(0, K2 // 16, group, 0)
            pltpu.sync_copy(msg, acc.at[idx_d], add=True)
            return carry

        lax.fori_loop(0, NCH2, chunk, 0)
    plsc.subcore_barrier()
    h0 = sid * SEL_PT
    for cc in range(SEL_PT // SELC):
        hb = h0 + cc * SELC
        pltpu.sync_copy(sel_idx.at[pl.ds(hb, SELC)], selidx)
        pltpu.async_copy(acc.at[selidx], selrows, s2).wait()
        pltpu.sync_copy(selrows, sel_out.at[pl.ds(hb, SELC)])
        if extra is not None:
            disc_t, disc_e = extra
            pltpu.async_copy(disc_t.at[selidx], drows, s2).wait()
            pltpu.sync_copy(drows, disc_e.at[pl.ds(hb, SELC)])


def _p2_body(u1, i1, u0, i0, v_ui1, v_ui0, v_iu1, v_iu0,
             ue1, ie1, stu_id, exer_id, disc_t,
             stu_e, ex_e, disc_e,
             idx_d, idx_s, vbuf, src_rows, msg, selidx, selrows, drows,
             acc, s1, s2):
    cid = lax.axis_index("c")
    sid = lax.axis_index("s")
    bufs = (idx_d, idx_s, vbuf, src_rows, msg, selidx, selrows, drows, s1, s2)

    @pl.when(cid == 0)
    def _():
        polars = ((u1, i1, v_ui1), (u0, i0, v_ui0))
        _p2_side(sid, polars, ue1, ie1, stu_id, stu_e, acc, bufs)

    @pl.when(cid == 1)
    def _():
        polars = ((i1, u1, v_iu1), (i0, u0, v_iu0))
        _p2_side(sid, polars, ie1, ue1, exer_id, ex_e, acc, bufs,
                 extra=(disc_t, disc_e))


def _pass2(u1, i1, u0, i0, vs, ue1, ie1, stu_id, exer_id, disc_t):
    return pl.kernel(
        _p2_body,
        out_type=[_f32((NB, DIM)), _f32((NB, DIM)), _f32((NB,))],
        mesh=_mesh,
        scratch_types=[
            pltpu.VMEM((K2,), jnp.int32),
            pltpu.VMEM((K2,), jnp.int32),
            pltpu.VMEM((K2,), jnp.float32),
            pltpu.VMEM((K2, DIM), jnp.float32),
            pltpu.VMEM((K2, DIM), jnp.float32),
            pltpu.VMEM((SELC,), jnp.int32),
            pltpu.VMEM((SELC, DIM), jnp.float32),
            pltpu.VMEM((SELC,), jnp.float32),
            pltpu.VMEM_SHARED((SN, DIM), jnp.float32),
            pltpu.SemaphoreType.DMA,
            pltpu.SemaphoreType.DMA,
        ],
    )(u1, i1, u0, i0, *vs, ue1, ie1, stu_id, exer_id, disc_t)


def _dotT(x, w):
    return lax.dot_general(x, w, (((1,), (1,)), ((), ())),
                           preferred_element_type=jnp.float32)


def _prep_tables_body(sw, ew, w11, b11, w01, b01,
                      d1u, s1i, d1i, s1u, d0u, s0i, d0i, s0u):
    swv = sw[...]
    ewv = ew[...]
    for wref, bref, du, si_, di, su_ in (
            (w11, b11, d1u, s1i, d1i, s1u), (w01, b01, d0u, s0i, d0i, s0u)):
        w = wref[...]
        a = _dotT(swv, w[:, :DIM]) + bref[...]
        b = _dotT(ewv, w[:, DIM:])
        du[...] = a
        di[...] = b
        si_[:, :DIM] = b
        si_[:, DIM:] = ewv
        su_[:, :DIM] = a
        su_[:, DIM:] = swv


def _prep_tables(sW, eW, l1_W1, l1_b1, l0_W1, l0_b1):
    R = 1000
    grid = SN // R
    blk = pl.BlockSpec((R, DIM), lambda i: (i, 0))
    blk2 = pl.BlockSpec((R, 2 * DIM), lambda i: (i, 0))
    full = lambda s: pl.BlockSpec(s, lambda i: tuple(0 for _ in s))
    return pl.pallas_call(
        _prep_tables_body,
        grid=(grid,),
        in_specs=[blk, blk, full((DIM, 2 * DIM)), full((1, DIM)),
                  full((DIM, 2 * DIM)), full((1, DIM))],
        out_specs=[blk, blk2, blk, blk2, blk, blk2, blk, blk2],
        out_shape=[_f32((SN, DIM)), _f32((EN, 2 * DIM)),
                   _f32((EN, DIM)), _f32((SN, 2 * DIM)),
                   _f32((SN, DIM)), _f32((EN, 2 * DIM)),
                   _f32((EN, DIM)), _f32((SN, 2 * DIM))],
    )(sW, eW, l1_W1, l1_b1.reshape(1, DIM), l0_W1, l0_b1.reshape(1, DIM))


def _prep_small_body(degp, edisc, su4, disc):
    d = degp[...]
    su4[...] = lax.rsqrt(d[0] + d[1] + 1.0)
    disc[...] = jax.nn.sigmoid(edisc[...]) * 10.0


def _prep_small(degp, e_disc_W):
    return pl.pallas_call(
        _prep_small_body,
        out_shape=[_f32((4, SN)), _f32((1, EN))],
    )(degp, e_disc_W)


def _prep_edge_body(e1, e2, e3, e4, b21, b20, o1, o2, o3, o4):
    def lg(x, b2):
        xc = jnp.clip(x, 1e-6, 1.0 - 1e-6)
        return 5.0 * (jnp.log(xc) - jnp.log1p(-xc) + b2)
    b1v = b21[0, 0]
    b0v = b20[0, 0]
    o1[...] = lg(e1[...], b1v)
    o2[...] = lg(e2[...], b1v)
    o3[...] = lg(e3[...], b0v)
    o4[...] = lg(e4[...], b0v)


def _prep_edge(eps_ui_1, eps_iu_1, eps_ui_0, eps_iu_0, l1_b2, l0_b2):
    W = 128
    rs = lambda x: x.reshape(E // W, W)
    outs = pl.pallas_call(
        _prep_edge_body,
        out_shape=[_f32((E // W, W))] * 4,
    )(rs(eps_ui_1), rs(eps_iu_1), rs(eps_ui_0), rs(eps_iu_0),
      l1_b2.reshape(1, 1), l0_b2.reshape(1, 1))
    return [o.reshape(E) for o in outs]


def _head_body(stu, ex, disc, knr, knw, p1w, p1b, p2w, p2b, p3w, p3b, out):
    stat = jax.nn.sigmoid(_dotT(stu[...], knw[...]))
    kdiff = jax.nn.sigmoid(_dotT(ex[...], knw[...]))
    x = disc[...] * (stat - kdiff) * knr[...]
    h1 = jax.nn.sigmoid(_dotT(x, jnp.abs(p1w[...])) + p1b[...])
    h2 = jax.nn.sigmoid(_dotT(h1, jnp.abs(p2w[...])) + p2b[...])
    w3 = jnp.concatenate([jnp.abs(p3w[...]), jnp.zeros((127, DIM), jnp.float32)],
                         axis=0)
    r = _dotT(h2, w3)
    out[...] = jax.nn.sigmoid(r[:, 0:1] + p3b[0, 0])


def _head(stu_e, ex_e, disc_e, kn_r, knowledge_W,
          pn1_W, pn1_b, pn2_W, pn2_b, pn3_W, pn3_b):
    R = 512
    grid = NB // R
    blk = pl.BlockSpec((R, DIM), lambda i: (i, 0))
    blk1 = pl.BlockSpec((R, 1), lambda i: (i, 0))
    full = lambda s: pl.BlockSpec(s, lambda i: tuple(0 for _ in s))
    out = pl.pallas_call(
        _head_body,
        grid=(grid,),
        in_specs=[blk, blk, blk1, blk,
                  full((DIM, DIM)),
                  full((256, DIM)), full((1, 256)),
                  full((DIM, 256)), full((1, DIM)),
                  full((1, DIM)), full((1, 1))],
        out_specs=blk1,
        out_shape=_f32((NB, 1)),
    )(stu_e, ex_e, disc_e.reshape(NB, 1), kn_r, knowledge_W,
      pn1_W, pn1_b.reshape(1, 256), pn2_W, pn2_b.reshape(1, DIM),
      pn3_W, pn3_b.reshape(1, 1))
    return out.reshape(NB)


def kernel(stu_id, exer_id, kn_r, edge_index_1, edge_index_0,
           eps_ui_1, eps_iu_1, eps_ui_0, eps_iu_0,
           student_W, exercise_W, knowledge_W, e_disc_W,
           l1_W1, l1_b1, l1_W2, l1_b2,
           l0_W1, l0_b1, l0_W2, l0_b2,
           pn1_W, pn1_b, pn2_W, pn2_b, pn3_W, pn3_b):
    u1 = edge_index_1[0]
    i1 = edge_index_1[1]
    u0 = edge_index_0[0]
    i0 = edge_index_0[1]

    degp = _pass0(u1, i1, u0, i0)
    su4, disc = _prep_small(degp.reshape(2, 4, SN), e_disc_W.reshape(1, EN))
    su1, si1, su0, si0 = su4[0], su4[1], su4[2], su4[3]
    disc_t = disc.reshape(EN)

    tabs = _prep_tables(student_W, exercise_W, l1_W1, l1_b1, l0_W1, l0_b1)
    lgs = _prep_edge(eps_ui_1, eps_iu_1, eps_ui_0, eps_iu_0, l1_b2, l0_b2)

    w2s = jnp.zeros((16, DIM), jnp.float32)
    w2s = w2s.at[0].set(l1_W2[0] * 5.0).at[8].set(l0_W2[0] * 5.0)

    ue1, ie1, v_ui1, v_ui0, v_iu1, v_iu0 = _pass1(
        u1, i1, u0, i0, lgs, tabs, (su1, si1, su0, si0), w2s,
        student_W, exercise_W)

    stu_e, ex_e, disc_e = _pass2(
        u1, i1, u0, i0, (v_ui1, v_ui0, v_iu1, v_iu0),
        ue1, ie1, stu_id, exer_id, disc_t)

    return _head(stu_e, ex_e, disc_e, kn_r, knowledge_W,
                 pn1_W, pn1_b, pn2_W, pn2_b, pn3_W, pn3_b)


# speedup vs baseline: 7.4727x; 2.2686x over previous
import jax
import jax.numpy as jnp
from jax import lax
from jax.experimental import pallas as pl
from jax.experimental.pallas import tpu as pltpu
from jax.experimental.pallas import tpu_sc as plsc

SN = 10000
EN = 10000
DIM = 128
E = 320000
NB = 4096
CONTROL = 0.3
NT = 16
EPT = E // NT
K1 = 80
NCH1 = EPT // K1
K2 = 80
NCH2 = EPT // K2
K0 = 2000
NCH0 = (E // 8) // K0
SEL_PT = NB // NT
SELC = 64
DW = DIM // 2
ROWS_SPLIT = [(t * 632, 632) for t in range(15)] + [(9480, 520)]

_mesh = plsc.VectorSubcoreMesh(core_axis_name="c", subcore_axis_name="s")


def _f32(shape):
    return jax.ShapeDtypeStruct(shape, jnp.float32)


def _i32(shape):
    return jax.ShapeDtypeStruct(shape, jnp.int32)


def _takev(v, idx):
    return v.at[idx].get(mode="promise_in_bounds")


def _lo(x):
    return lax.bitcast_convert_type(lax.shift_left(x, 16), jnp.float32)


def _hi(x):
    return lax.bitcast_convert_type(jnp.bitwise_and(x, jnp.int32(-65536)),
                                    jnp.float32)


def _rows_io(sid, src, dst):
    for t, (b, s) in enumerate(ROWS_SPLIT):
        @pl.when(sid == t)
        def _(b=b, s=s):
            pltpu.sync_copy(src.at[pl.ds(b, s)], dst.at[pl.ds(b, s)])


def _hsum16(accs):
    lane = lax.iota(jnp.int32, 16)
    cur = accs
    for fold, rotk in ((8, 8), (4, 12), (2, 14), (1, 15)):
        fidx = jnp.bitwise_and(lane + fold, 15)
        ridx = jnp.bitwise_and(lane + rotk, 15)
        m = jnp.bitwise_and(lane, 2 * fold - 1) < fold
        nxt = []
        for a, b in zip(cur[0::2], cur[1::2]):
            fa = a + _takev(a, fidx)
            fb = b + _takev(b, fidx)
            nxt.append(jnp.where(m, fa, _takev(fb, ridx)))
        cur = nxt
    perm = (jnp.bitwise_and(lane, 1) * 8 + jnp.bitwise_and(lane, 2) * 2
            + lax.shift_right_logical(jnp.bitwise_and(lane, 4), 1)
            + lax.shift_right_logical(jnp.bitwise_and(lane, 8), 3))
    return _takev(cur[0], perm)


def _p0_body(u1, i1, u0, i0, ones_h, degp,
             idxb, onesb, zbuf, a0, a1, a2, a3):
    cid = lax.axis_index("c")
    sid = lax.axis_index("s")
    accs = (a0, a1, a2, a3)
    arrays = (u1, i1, u0, i0)

    @pl.when(sid == 0)
    def _():
        def zfill(i, c):
            zbuf[pl.ds(i * 16, 16)] = jnp.zeros((16,), jnp.float32)
            return c
        lax.fori_loop(0, SN // 16, zfill, 0)
        for k in range(4):
            pltpu.sync_copy(zbuf, accs[k])

    pltpu.sync_copy(ones_h, onesb)
    plsc.subcore_barrier()

    a = sid // 4
    q = jnp.remainder(sid, 4)
    wrk = cid * 4 + q

    def chunk(c, carry):
        base = wrk * (E // 8) + c * K0
        for k in range(4):
            @pl.when(a == k)
            def _(k=k):
                pltpu.sync_copy(arrays[k].at[pl.ds(base, K0)], idxb)
                pltpu.sync_copy(onesb, accs[k].at[idxb], add=True)
        return carry

    lax.fori_loop(0, NCH0, chunk, 0)
    plsc.subcore_barrier()
    for k in range(4):
        @pl.when(sid == k)
        def _(k=k):
            pltpu.sync_copy(accs[k], zbuf)
            pltpu.sync_copy(zbuf, degp.at[pl.ds((cid * 4 + k) * SN, SN)])


def _pass0(u1, i1, u0, i0):
    ones_h = jnp.ones((K0,), jnp.float32)
    return pl.kernel(
        _p0_body,
        out_type=_f32((8 * SN,)),
        mesh=_mesh,
        scratch_types=[
            pltpu.VMEM((K0,), jnp.int32),
            pltpu.VMEM((K0,), jnp.float32),
            pltpu.VMEM((SN,), jnp.float32),
            pltpu.VMEM_SHARED((SN,), jnp.float32),
            pltpu.VMEM_SHARED((SN,), jnp.float32),
            pltpu.VMEM_SHARED((SN,), jnp.float32),
            pltpu.VMEM_SHARED((SN,), jnp.float32),
        ],
    )(u1, i1, u0, i0, ones_h)


EPT2 = 2 * EPT
NCHT = EPT2 // K1
NCHT2 = EPT2 // K2


def _p1_gate(s, bufs, w2v):
    (ius, irs, its, iss, lgs_, dstb, srcs, sus, sis, msg, vouts) = bufs
    lg, src, vout = lgs_[s], srcs[s], vouts[s]
    su_e, si_e = sus[s], sis[s]

    def group(g, carry):
        e0 = g * 16
        accs = []
        for j in range(16):
            e = e0 + j
            acc16 = None
            for w in range(4):
                dI = dstb[e, pl.ds(16 * w, 16)]
                sI = src[e, pl.ds(16 * w, 16)]
                hl = jnp.maximum(_lo(dI) + _lo(sI), 0.0)
                hh = jnp.maximum(_hi(dI) + _hi(sI), 0.0)
                t = hl * w2v[2 * w] + hh * w2v[2 * w + 1]
                acc16 = t if acc16 is None else acc16 + t
            accs.append(acc16)
        logit = _hsum16(accs)
        gv = logit + lg[pl.ds(e0, 16)]
        wgt = CONTROL / (1.0 + jnp.exp(-gv)) + (1.0 - CONTROL)
        v = su_e[pl.ds(e0, 16)] * si_e[pl.ds(e0, 16)] * wgt
        vout[pl.ds(e0, 16)] = v
        return carry

    lax.fori_loop(0, K1 // 16, group, 0)


def _p1_msg(s, bufs, acc):
    (ius, irs, its, iss, lgs_, dstb, srcs, sus, sis, msg, vouts) = bufs
    src, vout = srcs[s], vouts[s]

    def group(g, carry):
        e0 = g * 16
        v16 = vout[pl.ds(e0, 16)]
        for j in range(16):
            e = e0 + j
            vs = _takev(v16, jnp.full((16,), j, jnp.int32))
            for w in range(4):
                mI = src[e, pl.ds(DW + 16 * w, 16)]
                msg[e, pl.ds(32 * w, 16)] = vs * _lo(mI)
                msg[e, pl.ds(32 * w + 16, 16)] = vs * _hi(mI)
        return carry

    lax.fori_loop(0, K1 // 16, group, 0)
    pltpu.sync_copy(msg, acc.at[ius[s]], add=True)


def _p1_section(sid, hbm, bufs, sems, w2a, w2b, acc):
    (dr_h, sr_h, lg_h, dstT, srcT, su_t, si_t, v_h) = hbm
    (ius, irs, its, iss, lgs_, dstb, srcs, sus, sis, msg, vouts) = bufs
    (sA, sB, sD, sV) = sems

    def issue_idx(c, s):
        base = sid * EPT2 + c * K1
        pltpu.async_copy(dr_h.at[pl.ds(base, K1)], ius[s], sA[s])
        pltpu.async_copy(sr_h.at[pl.ds(base, K1)], irs[s], sA[s])
        pltpu.async_copy(lg_h.at[pl.ds(base, K1)], lgs_[s], sA[s])

    def wait_idx(c, s):
        base = sid * EPT2 + c * K1
        pltpu.make_async_copy(dr_h.at[pl.ds(base, K1)], ius[s], sA[s]).wait()
        pltpu.make_async_copy(sr_h.at[pl.ds(base, K1)], irs[s], sA[s]).wait()
        pltpu.make_async_copy(lg_h.at[pl.ds(base, K1)], lgs_[s], sA[s]).wait()

    def mk_tabs(c, s):
        offv = jnp.where(c < NCHT // 2, jnp.zeros((16,), jnp.int32),
                         jnp.full((16,), SN, jnp.int32))
        for g in range(K1 // 16):
            e0 = g * 16
            its[s][pl.ds(e0, 16)] = ius[s][pl.ds(e0, 16)] + offv
            iss[s][pl.ds(e0, 16)] = irs[s][pl.ds(e0, 16)] + offv

    def issue_src(c, s):
        pltpu.async_copy(srcT.at[iss[s]], srcs[s], sB[s])
        pltpu.async_copy(su_t.at[its[s]], sus[s], sB[s])
        pltpu.async_copy(si_t.at[iss[s]], sis[s], sB[s])

    def wait_src(c, s):
        pltpu.make_async_copy(srcT.at[iss[s]], srcs[s], sB[s]).wait()
        pltpu.make_async_copy(su_t.at[its[s]], sus[s], sB[s]).wait()
        pltpu.make_async_copy(si_t.at[iss[s]], sis[s], sB[s]).wait()

    def issue_dst(s):
        pltpu.async_copy(dstT.at[its[s]], dstb, sD)

    def wait_dst(s):
        pltpu.make_async_copy(dstT.at[its[s]], dstb, sD).wait()

    def issue_vout(c, s):
        base = sid * EPT2 + c * K1
        pltpu.async_copy(vouts[s], v_h.at[pl.ds(base, K1)], sV[s])

    def wait_vout(c, s):
        base = sid * EPT2 + c * K1
        pltpu.make_async_copy(vouts[s], v_h.at[pl.ds(base, K1)], sV[s]).wait()

    def handler(c, s):
        w2v = [jnp.where(c < NCHT // 2, w2a[f], w2b[f]) for f in range(8)]
        wait_src(c, s)
        wait_dst(s)

        @pl.when(c + 1 < NCHT)
        def _():
            wait_idx(c + 1, 1 - s)
            mk_tabs(c + 1, 1 - s)
            issue_src(c + 1, 1 - s)

        @pl.when(c >= 2)
        def _():
            wait_vout(c - 2, s)

        _p1_gate(s, bufs, w2v)

        @pl.when(c + 1 < NCHT)
        def _():
            issue_dst(1 - s)

        _p1_msg(s, bufs, acc)
        issue_vout(c, s)

        @pl.when(c + 2 < NCHT)
        def _():
            issue_idx(c + 2, s)

    issue_idx(0, 0)
    wait_idx(0, 0)
    mk_tabs(0, 0)
    issue_src(0, 0)
    issue_dst(0)
    issue_idx(1, 1)

    def pair(c2, carry):
        handler(2 * c2, 0)
        handler(2 * c2 + 1, 1)
        return carry

    lax.fori_loop(0, NCHT // 2, pair, 0)
    wait_vout(NCHT - 2, (NCHT - 2) % 2)
    wait_vout(NCHT - 1, (NCHT - 1) % 2)


def _p1_side(sid, hbm, init_tab, out_tab, acc, bufs, sems, w2buf, w2s):
    _rows_io(sid, init_tab, acc)
    pltpu.sync_copy(w2s, w2buf)
    plsc.subcore_barrier()
    w2a = [w2buf[0, pl.ds(16 * f, 16)] for f in range(8)]
    w2b = [w2buf[8, pl.ds(16 * f, 16)] for f in range(8)]
    _p1_section(sid, hbm, bufs, sems, w2a, w2b, acc)
    plsc.subcore_barrier()
    _rows_io(sid, acc, out_tab)


def _p1_body(u_raw, i_raw, lg_ui, lg_iu,
             tu, ti, su_a, si_a, w2s, sW, eW,
             ue1, ie1, v_ui, v_iu,
             iu0b, iu1b, ir0b, ir1b, it0b, it1b, is0b, is1b, lg0b, lg1b,
             dstb, src0b, src1b, su0b, su1b, si0b, si1b,
             msg, vo0b, vo1b,
             w2buf, acc, sA0, sA1, sB0, sB1, sD, sV0, sV1):
    cid = lax.axis_index("c")
    sid = lax.axis_index("s")
    bufs = ((iu0b, iu1b), (ir0b, ir1b), (it0b, it1b), (is0b, is1b),
            (lg0b, lg1b), dstb,
            (src0b, src1b), (su0b, su1b), (si0b, si1b), msg,
            (vo0b, vo1b))
    sems = ((sA0, sA1), (sB0, sB1), sD, (sV0, sV1))

    @pl.when(cid == 0)
    def _():
        hbm = (u_raw, i_raw, lg_ui, tu, ti, su_a, si_a, v_ui)
        _p1_side(sid, hbm, sW, ue1, acc, bufs, sems, w2buf, w2s)

    @pl.when(cid == 1)
    def _():
        hbm = (i_raw, u_raw, lg_iu, ti, tu, si_a, su_a, v_iu)
        _p1_side(sid, hbm, eW, ie1, acc, bufs, sems, w2buf, w2s)


def _pass1(u_raw, i_raw, lg_ui, lg_iu,
           tu, ti, su_a, si_a, w2s, sW, eW):
    return pl.kernel(
        _p1_body,
        out_type=[_f32((SN, DIM)), _f32((EN, DIM)),
                  _f32((2 * E,)), _f32((2 * E,))],
        mesh=_mesh,
        scratch_types=[
            pltpu.VMEM((K1,), jnp.int32),
            pltpu.VMEM((K1,), jnp.int32),
            pltpu.VMEM((K1,), jnp.int32),
            pltpu.VMEM((K1,), jnp.int32),
            pltpu.VMEM((K1,), jnp.int32),
            pltpu.VMEM((K1,), jnp.int32),
            pltpu.VMEM((K1,), jnp.int32),
            pltpu.VMEM((K1,), jnp.int32),
            pltpu.VMEM((K1,), jnp.float32),
            pltpu.VMEM((K1,), jnp.float32),
            pltpu.VMEM((K1, 2 * DW), jnp.int32),
            pltpu.VMEM((K1, 2 * DW), jnp.int32),
            pltpu.VMEM((K1, 2 * DW), jnp.int32),
            pltpu.VMEM((K1,), jnp.float32),
            pltpu.VMEM((K1,), jnp.float32),
            pltpu.VMEM((K1,), jnp.float32),
            pltpu.VMEM((K1,), jnp.float32),
            pltpu.VMEM((K1, DIM), jnp.float32),
            pltpu.VMEM((K1,), jnp.float32),
            pltpu.VMEM((K1,), jnp.float32),
            pltpu.VMEM((16, DIM), jnp.float32),
            pltpu.VMEM_SHARED((SN, DIM), jnp.float32),
            pltpu.SemaphoreType.DMA,
            pltpu.SemaphoreType.DMA,
            pltpu.SemaphoreType.DMA,
            pltpu.SemaphoreType.DMA,
            pltpu.SemaphoreType.DMA,
            pltpu.SemaphoreType.DMA,
            pltpu.SemaphoreType.DMA,
        ],
    )(u_raw, i_raw, lg_ui, lg_iu, tu, ti, su_a, si_a,
      w2s, sW, eW)


def _p2_compute(s, bufs, acc):
    (ius, iis, vbs, srcs, msg) = bufs
    iu, vb, src = ius[s], vbs[s], srcs[s]

    def group(g, carry):
        e0 = g * 16
        v16 = vb[pl.ds(e0, 16)]
        for j in range(16):
            e = e0 + j
            vs = _takev(v16, jnp.full((16,), j, jnp.int32))
            for w in range(8):
                msg[e, pl.ds(16 * w, 16)] = vs * src[e, pl.ds(16 * w, 16)]
        return carry

    lax.fori_loop(0, K2 // 16, group, 0)
    pltpu.sync_copy(msg, acc.at[ius[s]], add=True)


def _p2_section(sid, hbm, bufs, sems, acc):
    (d_h, s_h, v_h, gatT) = hbm
    (ius, iis, vbs, srcs, msg) = bufs
    (sA, sB) = sems

    def issue_idx(c, s):
        base = sid * EPT2 + c * K2
        pltpu.async_copy(d_h.at[pl.ds(base, K2)], ius[s], sA[s])
        pltpu.async_copy(s_h.at[pl.ds(base, K2)], iis[s], sA[s])
        pltpu.async_copy(v_h.at[pl.ds(base, K2)], vbs[s], sA[s])

    def wait_idx(c, s):
        base = sid * EPT2 + c * K2
        pltpu.make_async_copy(d_h.at[pl.ds(base, K2)], ius[s], sA[s]).wait()
        pltpu.make_async_copy(s_h.at[pl.ds(base, K2)], iis[s], sA[s]).wait()
        pltpu.make_async_copy(v_h.at[pl.ds(base, K2)], vbs[s], sA[s]).wait()

    def issue_rows(c, s):
        pltpu.async_copy(gatT.at[iis[s]], srcs[s], sB[s])

    def wait_rows(c, s):
        pltpu.make_async_copy(gatT.at[iis[s]], srcs[s], sB[s]).wait()

    def handler(c, s):
        wait_rows(c, s)

        @pl.when(c + 1 < NCHT2)
        def _():
            wait_idx(c + 1, 1 - s)
            issue_rows(c + 1, 1 - s)

        _p2_compute(s, bufs, acc)

        @pl.when(c + 2 < NCHT2)
        def _():
            issue_idx(c + 2, s)

    issue_idx(0, 0)
    wait_idx(0, 0)
    issue_rows(0, 0)
    issue_idx(1, 1)

    def pair(c2, carry):
        handler(2 * c2, 0)
        handler(2 * c2 + 1, 1)
        return carry

    lax.fori_loop(0, NCHT2 // 2, pair, 0)


def _p2_side(sid, hbm, init_tab, sel_idx, sel_out, acc, bufs, sems,
             selbufs, extra=None):
    (selidx, selrows, drows, sS) = selbufs
    _rows_io(sid, init_tab, acc)
    plsc.subcore_barrier()
    _p2_section(sid, hbm, bufs, sems, acc)
    plsc.subcore_barrier()
    h0 = sid * SEL_PT
    for cc in range(SEL_PT // SELC):
        hb = h0 + cc * SELC
        pltpu.sync_copy(sel_idx.at[pl.ds(hb, SELC)], selidx)
        pltpu.async_copy(acc.at[selidx], selrows, sS).wait()
        pltpu.sync_copy(selrows, sel_out.at[pl.ds(hb, SELC)])
        if extra is not None:
            disc_t, disc_e = extra
            pltpu.async_copy(disc_t.at[selidx], drows, sS).wait()
            pltpu.sync_copy(drows, disc_e.at[pl.ds(hb, SELC)])


def _p2_body(u_raw, i_raw, v_ui, v_iu,
             ue1, ie1, stu_id, exer_id, disc_t,
             stu_e, ex_e, disc_e,
             iu0b, iu1b, ii0b, ii1b, vb0, vb1, src0b, src1b, msg,
             selidx, selrows, drows,
             acc, sA0, sA1, sB0, sB1, sS):
    cid = lax.axis_index("c")
    sid = lax.axis_index("s")
    bufs = ((iu0b, iu1b), (ii0b, ii1b), (vb0, vb1), (src0b, src1b), msg)
    sems = ((sA0, sA1), (sB0, sB1))
    selbufs = (selidx, selrows, drows, sS)

    @pl.when(cid == 0)
    def _():
        hbm = (u_raw, i_raw, v_ui, ie1)
        _p2_side(sid, hbm, ue1, stu_id, stu_e, acc, bufs, sems, selbufs)

    @pl.when(cid == 1)
    def _():
        hbm = (i_raw, u_raw, v_iu, ue1)
        _p2_side(sid, hbm, ie1, exer_id, ex_e, acc, bufs, sems, selbufs,
                 extra=(disc_t, disc_e))


def _pass2(u_raw, i_raw, v_ui, v_iu, ue1, ie1,
           stu_id, exer_id, disc_t):
    return pl.kernel(
        _p2_body,
        out_type=[_f32((NB, DIM)), _f32((NB, DIM)), _f32((NB,))],
        mesh=_mesh,
        scratch_types=[
            pltpu.VMEM((K2,), jnp.int32),
            pltpu.VMEM((K2,), jnp.int32),
            pltpu.VMEM((K2,), jnp.int32),
            pltpu.VMEM((K2,), jnp.int32),
            pltpu.VMEM((K2,), jnp.float32),
            pltpu.VMEM((K2,), jnp.float32),
            pltpu.VMEM((K2, DIM), jnp.float32),
            pltpu.VMEM((K2, DIM), jnp.float32),
            pltpu.VMEM((K2, DIM), jnp.float32),
            pltpu.VMEM((SELC,), jnp.int32),
            pltpu.VMEM((SELC, DIM), jnp.float32),
            pltpu.VMEM((SELC,), jnp.float32),
            pltpu.VMEM_SHARED((SN, DIM), jnp.float32),
            pltpu.SemaphoreType.DMA,
            pltpu.SemaphoreType.DMA,
            pltpu.SemaphoreType.DMA,
            pltpu.SemaphoreType.DMA,
            pltpu.SemaphoreType.DMA,
        ],
    )(u_raw, i_raw, v_ui, v_iu, ue1, ie1,
      stu_id, exer_id, disc_t)


def _dotT(x, w):
    return lax.dot_general(x, w, (((1,), (1,)), ((), ())),
                           preferred_element_type=jnp.float32)


def _prep_tables_body(sw, ew, w11, b11, w01, b01,
                      s1u, s1i, s0u, s0i):
    swv = sw[...]
    ewv = ew[...]
    swb = swv.astype(jnp.bfloat16)
    ewb = ewv.astype(jnp.bfloat16)
    for wref, bref, su_, si_ in (
            (w11, b11, s1u, s1i), (w01, b01, s0u, s0i)):
        w = wref[...]
        a = (_dotT(swv, w[:, :DIM]) + bref[...]).astype(jnp.bfloat16)
        b = (_dotT(ewv, w[:, DIM:])).astype(jnp.bfloat16)
        su_[...] = jnp.concatenate([a, swb], axis=1)
        si_[...] = jnp.concatenate([b, ewb], axis=1)


def _prep_tables(sW, eW, l1_W1, l1_b1, l0_W1, l0_b1):
    R = 1000
    grid = SN // R
    blk = pl.BlockSpec((R, DIM), lambda i: (i, 0))
    blk2 = pl.BlockSpec((R, 2 * DIM), lambda i: (i, 0))
    full = lambda s: pl.BlockSpec(s, lambda i: tuple(0 for _ in s))
    bf = lambda s: jax.ShapeDtypeStruct(s, jnp.bfloat16)
    return pl.pallas_call(
        _prep_tables_body,
        grid=(grid,),
        in_specs=[blk, blk, full((DIM, 2 * DIM)), full((1, DIM)),
                  full((DIM, 2 * DIM)), full((1, DIM))],
        out_specs=[blk2, blk2, blk2, blk2],
        out_shape=[bf((SN, 2 * DIM)), bf((EN, 2 * DIM)),
                   bf((SN, 2 * DIM)), bf((EN, 2 * DIM))],
    )(sW, eW, l1_W1, l1_b1.reshape(1, DIM), l0_W1, l0_b1.reshape(1, DIM))


def _prep_small_body(degp, edisc, su4, disc):
    d = degp[...]
    su4[...] = lax.rsqrt(d[0] + d[1] + 1.0)
    disc[...] = jax.nn.sigmoid(edisc[...]) * 10.0


def _prep_small(degp, e_disc_W):
    return pl.pallas_call(
        _prep_small_body,
        out_shape=[_f32((4, SN)), _f32((1, EN))],
    )(degp, e_disc_W)


def _prep_edge_body(e1, e2, e3, e4, b21, b20, o1, o2, o3, o4):
    def lg(x, b2):
        xc = jnp.clip(x, 1e-6, 1.0 - 1e-6)
        return 5.0 * (jnp.log(xc) - jnp.log1p(-xc) + b2)
    b1v = b21[0, 0]
    b0v = b20[0, 0]
    o1[...] = lg(e1[...], b1v)
    o2[...] = lg(e2[...], b1v)
    o3[...] = lg(e3[...], b0v)
    o4[...] = lg(e4[...], b0v)


def _prep_edge(eps_ui_1, eps_iu_1, eps_ui_0, eps_iu_0, l1_b2, l0_b2):
    W = 128
    rs = lambda x: x.reshape(E // W, W)
    outs = pl.pallas_call(
        _prep_edge_body,
        out_shape=[_f32((E // W, W))] * 4,
    )(rs(eps_ui_1), rs(eps_iu_1), rs(eps_ui_0), rs(eps_iu_0),
      l1_b2.reshape(1, 1), l0_b2.reshape(1, 1))
    return [o.reshape(E) for o in outs]


def _head_body(stu, ex, disc, knr, knw, p1w, p1b, p2w, p2b, p3w, p3b, out):
    stat = jax.nn.sigmoid(_dotT(stu[...], knw[...]))
    kdiff = jax.nn.sigmoid(_dotT(ex[...], knw[...]))
    x = disc[...] * (stat - kdiff) * knr[...]
    h1 = jax.nn.sigmoid(_dotT(x, jnp.abs(p1w[...])) + p1b[...])
    h2 = jax.nn.sigmoid(_dotT(h1, jnp.abs(p2w[...])) + p2b[...])
    w3 = jnp.concatenate([jnp.abs(p3w[...]), jnp.zeros((127, DIM), jnp.float32)],
                         axis=0)
    r = _dotT(h2, w3)
    out[...] = jax.nn.sigmoid(r[:, 0:1] + p3b[0, 0])


def _head(stu_e, ex_e, disc_e, kn_r, knowledge_W,
          pn1_W, pn1_b, pn2_W, pn2_b, pn3_W, pn3_b):
    R = 512
    grid = NB // R
    blk = pl.BlockSpec((R, DIM), lambda i: (i, 0))
    blk1 = pl.BlockSpec((R, 1), lambda i: (i, 0))
    full = lambda s: pl.BlockSpec(s, lambda i: tuple(0 for _ in s))
    out = pl.pallas_call(
        _head_body,
        grid=(grid,),
        in_specs=[blk, blk, blk1, blk,
                  full((DIM, DIM)),
                  full((256, DIM)), full((1, 256)),
                  full((DIM, 256)), full((1, DIM)),
                  full((1, DIM)), full((1, 1))],
        out_specs=blk1,
        out_shape=_f32((NB, 1)),
    )(stu_e, ex_e, disc_e.reshape(NB, 1), kn_r, knowledge_W,
      pn1_W, pn1_b.reshape(1, 256), pn2_W, pn2_b.reshape(1, DIM),
      pn3_W, pn3_b.reshape(1, 1))
    return out.reshape(NB)


def _packw(t):
    n, f = t.shape
    arr = t.reshape(n, f // 32, 2, 16).transpose(0, 1, 3, 2)
    return lax.bitcast_convert_type(arr, jnp.int32).reshape(n, f // 2)


def _with_scalar(packed, s):
    si = lax.bitcast_convert_type(s.reshape(-1, 1), jnp.int32)
    z = jnp.zeros((packed.shape[0], 15), jnp.int32)
    return jnp.concatenate([packed, si, z], axis=1)


def kernel(stu_id, exer_id, kn_r, edge_index_1, edge_index_0,
           eps_ui_1, eps_iu_1, eps_ui_0, eps_iu_0,
           student_W, exercise_W, knowledge_W, e_disc_W,
           l1_W1, l1_b1, l1_W2, l1_b2,
           l0_W1, l0_b1, l0_W2, l0_b2,
           pn1_W, pn1_b, pn2_W, pn2_b, pn3_W, pn3_b):
    u1 = edge_index_1[0]
    i1 = edge_index_1[1]
    u0 = edge_index_0[0]
    i0 = edge_index_0[1]

    degp = _pass0(u1, i1, u0, i0)
    su4, disc = _prep_small(degp.reshape(2, 4, SN), e_disc_W.reshape(1, EN))
    su1, si1, su0, si0 = su4[0], su4[1], su4[2], su4[3]
    disc_t = disc.reshape(EN)

    s1u, s1i, s0u, s0i = _prep_tables(
        student_W, exercise_W, l1_W1, l1_b1, l0_W1, l0_b1)
    tu = jnp.concatenate([_packw(s1u), _packw(s0u)], axis=0)
    ti = jnp.concatenate([_packw(s1i), _packw(s0i)], axis=0)
    su_a = jnp.concatenate([su1, su0])
    si_a = jnp.concatenate([si1, si0])
    lgs = _prep_edge(eps_ui_1, eps_iu_1, eps_ui_0, eps_iu_0, l1_b2, l0_b2)

    tile = lambda x: x.reshape(NT, EPT)
    cat2 = lambda a, b: jnp.concatenate([tile(a), tile(b)], axis=1).reshape(-1)
    u_raw = cat2(u1, u0)
    i_raw = cat2(i1, i0)
    lg_ui = cat2(lgs[0], lgs[2])
    lg_iu = cat2(lgs[1], lgs[3])

    w2s = jnp.zeros((16, DIM), jnp.float32)
    w2s = w2s.at[0].set(l1_W2[0] * 5.0).at[8].set(l0_W2[0] * 5.0)

    ue1, ie1, v_ui, v_iu = _pass1(
        u_raw, i_raw, lg_ui, lg_iu,
        tu, ti, su_a, si_a, w2s, student_W, exercise_W)

    stu_e, ex_e, disc_e = _pass2(
        u_raw, i_raw, v_ui, v_iu, ue1, ie1, stu_id, exer_id, disc_t)

    return _head(stu_e, ex_e, disc_e, kn_r, knowledge_W,
                 pn1_W, pn1_b, pn2_W, pn2_b, pn3_W, pn3_b)


# speedup vs baseline: 7.4758x; 1.0004x over previous
import jax
import jax.numpy as jnp
from jax import lax
from jax.experimental import pallas as pl
from jax.experimental.pallas import tpu as pltpu
from jax.experimental.pallas import tpu_sc as plsc

SN = 10000
EN = 10000
DIM = 128
E = 320000
NB = 4096
CONTROL = 0.3
NT = 16
EPT = E // NT
K1 = 80
K2 = 80
K0 = 2000
NCH0 = (E // 8) // K0
SEL_PT = NB // NT
SELC = 64
DW = DIM // 2
ROWS_SPLIT = [(t * 632, 632) for t in range(15)] + [(9480, 520)]

_mesh = plsc.VectorSubcoreMesh(core_axis_name="c", subcore_axis_name="s")


def _f32(shape):
    return jax.ShapeDtypeStruct(shape, jnp.float32)


def _takev(v, idx):
    return v.at[idx].get(mode="promise_in_bounds")


def _lo(x):
    return lax.bitcast_convert_type(lax.shift_left(x, 16), jnp.float32)


def _hi(x):
    return lax.bitcast_convert_type(jnp.bitwise_and(x, jnp.int32(-65536)),
                                    jnp.float32)


def _rows_io(sid, src, dst):
    for t, (b, s) in enumerate(ROWS_SPLIT):
        @pl.when(sid == t)
        def _(b=b, s=s):
            pltpu.sync_copy(src.at[pl.ds(b, s)], dst.at[pl.ds(b, s)])


def _hsum16(accs):
    lane = lax.iota(jnp.int32, 16)
    cur = accs
    for fold, rotk in ((8, 8), (4, 12), (2, 14), (1, 15)):
        fidx = jnp.bitwise_and(lane + fold, 15)
        ridx = jnp.bitwise_and(lane + rotk, 15)
        m = jnp.bitwise_and(lane, 2 * fold - 1) < fold
        nxt = []
        for a, b in zip(cur[0::2], cur[1::2]):
            fa = a + _takev(a, fidx)
            fb = b + _takev(b, fidx)
            nxt.append(jnp.where(m, fa, _takev(fb, ridx)))
        cur = nxt
    perm = (jnp.bitwise_and(lane, 1) * 8 + jnp.bitwise_and(lane, 2) * 2
            + lax.shift_right_logical(jnp.bitwise_and(lane, 4), 1)
            + lax.shift_right_logical(jnp.bitwise_and(lane, 8), 3))
    return _takev(cur[0], perm)


def _p0_body(u1, i1, u0, i0, ones_h, degp,
             idxb, onesb, zbuf, a0, a1, a2, a3):
    cid = lax.axis_index("c")
    sid = lax.axis_index("s")
    accs = (a0, a1, a2, a3)
    arrays = (u1, i1, u0, i0)

    @pl.when(sid == 0)
    def _():
        def zfill(i, c):
            zbuf[pl.ds(i * 16, 16)] = jnp.zeros((16,), jnp.float32)
            return c
        lax.fori_loop(0, SN // 16, zfill, 0)
        for k in range(4):
            pltpu.sync_copy(zbuf, accs[k])

    pltpu.sync_copy(ones_h, onesb)
    plsc.subcore_barrier()

    a = sid // 4
    q = jnp.remainder(sid, 4)
    wrk = cid * 4 + q

    def chunk(c, carry):
        base = wrk * (E // 8) + c * K0
        for k in range(4):
            @pl.when(a == k)
            def _(k=k):
                pltpu.sync_copy(arrays[k].at[pl.ds(base, K0)], idxb)
                pltpu.sync_copy(onesb, accs[k].at[idxb], add=True)
        return carry

    lax.fori_loop(0, NCH0, chunk, 0)
    plsc.subcore_barrier()
    for k in range(4):
        @pl.when(sid == k)
        def _(k=k):
            pltpu.sync_copy(accs[k], zbuf)
            pltpu.sync_copy(zbuf, degp.at[pl.ds((cid * 4 + k) * SN, SN)])


def _pass0(u1, i1, u0, i0):
    ones_h = jnp.ones((K0,), jnp.float32)
    return pl.kernel(
        _p0_body,
        out_type=_f32((8 * SN,)),
        mesh=_mesh,
        scratch_types=[
            pltpu.VMEM((K0,), jnp.int32),
            pltpu.VMEM((K0,), jnp.float32),
            pltpu.VMEM((SN,), jnp.float32),
            pltpu.VMEM_SHARED((SN,), jnp.float32),
            pltpu.VMEM_SHARED((SN,), jnp.float32),
            pltpu.VMEM_SHARED((SN,), jnp.float32),
            pltpu.VMEM_SHARED((SN,), jnp.float32),
        ],
    )(u1, i1, u0, i0, ones_h)


EPT2 = 2 * EPT
NCHT = EPT2 // K1
NCHT2 = EPT2 // K2


def _p1_gate(s, bufs, w2v):
    (ius, irs, its, iss, lgs_, dstb, srcs, sus, sis, msg, vouts) = bufs
    lg, src, vout = lgs_[s], srcs[s], vouts[s]
    su_e, si_e = sus[s], sis[s]

    def group(g, carry):
        e0 = g * 16
        accs = []
        for j in range(16):
            e = e0 + j
            acc16 = None
            for w in range(4):
                dI = dstb[e, pl.ds(16 * w, 16)]
                sI = src[e, pl.ds(16 * w, 16)]
                hl = jnp.maximum(_lo(dI) + _lo(sI), 0.0)
                hh = jnp.maximum(_hi(dI) + _hi(sI), 0.0)
                t = hl * w2v[2 * w] + hh * w2v[2 * w + 1]
                acc16 = t if acc16 is None else acc16 + t
            accs.append(acc16)
        logit = _hsum16(accs)
        gv = logit + lg[pl.ds(e0, 16)]
        wgt = CONTROL / (1.0 + jnp.exp(-gv)) + (1.0 - CONTROL)
        v = su_e[pl.ds(e0, 16)] * si_e[pl.ds(e0, 16)] * wgt
        vout[pl.ds(e0, 16)] = v
        return carry

    lax.fori_loop(0, K1 // 16, group, 0)


def _p1_msg(s, bufs, acc):
    (ius, irs, its, iss, lgs_, dstb, srcs, sus, sis, msg, vouts) = bufs
    src, vout = srcs[s], vouts[s]

    def group(g, carry):
        e0 = g * 16
        v16 = vout[pl.ds(e0, 16)]
        for j in range(16):
            e = e0 + j
            vs = _takev(v16, jnp.full((16,), j, jnp.int32))
            for w in range(4):
                mI = src[e, pl.ds(DW + 16 * w, 16)]
                msg[e, pl.ds(32 * w, 16)] = vs * _lo(mI)
                msg[e, pl.ds(32 * w + 16, 16)] = vs * _hi(mI)
        return carry

    lax.fori_loop(0, K1 // 16, group, 0)
    pltpu.sync_copy(msg, acc.at[ius[s]], add=True)


def _p1_section(sid, hbm, bufs, sems, w2a, w2b, acc):
    (dr_h, sr_h, lg_h, dstT, srcT, su_t, si_t, v_h) = hbm
    (ius, irs, its, iss, lgs_, dstb, srcs, sus, sis, msg, vouts) = bufs
    (sA, sB, sD, sV) = sems

    def issue_idx(c, s):
        base = sid * EPT2 + c * K1
        pltpu.async_copy(dr_h.at[pl.ds(base, K1)], ius[s], sA[s])
        pltpu.async_copy(sr_h.at[pl.ds(base, K1)], irs[s], sA[s])
        pltpu.async_copy(lg_h.at[pl.ds(base, K1)], lgs_[s], sA[s])

    def wait_idx(c, s):
        base = sid * EPT2 + c * K1
        pltpu.make_async_copy(dr_h.at[pl.ds(base, K1)], ius[s], sA[s]).wait()
        pltpu.make_async_copy(sr_h.at[pl.ds(base, K1)], irs[s], sA[s]).wait()
        pltpu.make_async_copy(lg_h.at[pl.ds(base, K1)], lgs_[s], sA[s]).wait()

    def mk_tabs(c, s):
        offv = jnp.where(c < NCHT // 2, jnp.zeros((16,), jnp.int32),
                         jnp.full((16,), SN, jnp.int32))
        for g in range(K1 // 16):
            e0 = g * 16
            its[s][pl.ds(e0, 16)] = ius[s][pl.ds(e0, 16)] + offv
            iss[s][pl.ds(e0, 16)] = irs[s][pl.ds(e0, 16)] + offv

    def issue_src(c, s):
        pltpu.async_copy(srcT.at[iss[s]], srcs[s], sB[s])
        pltpu.async_copy(su_t.at[its[s]], sus[s], sB[s])
        pltpu.async_copy(si_t.at[iss[s]], sis[s], sB[s])

    def wait_src(c, s):
        pltpu.make_async_copy(srcT.at[iss[s]], srcs[s], sB[s]).wait()
        pltpu.make_async_copy(su_t.at[its[s]], sus[s], sB[s]).wait()
        pltpu.make_async_copy(si_t.at[iss[s]], sis[s], sB[s]).wait()

    def issue_dst(s):
        pltpu.async_copy(dstT.at[its[s]], dstb, sD)

    def wait_dst(s):
        pltpu.make_async_copy(dstT.at[its[s]], dstb, sD).wait()

    def issue_vout(c, s):
        base = sid * EPT2 + c * K1
        pltpu.async_copy(vouts[s], v_h.at[pl.ds(base, K1)], sV[s])

    def wait_vout(c, s):
        base = sid * EPT2 + c * K1
        pltpu.make_async_copy(vouts[s], v_h.at[pl.ds(base, K1)], sV[s]).wait()

    def handler(c, s):
        w2v = [jnp.where(c < NCHT // 2, w2a[f], w2b[f]) for f in range(8)]
        wait_src(c, s)
        wait_dst(s)

        @pl.when(c + 1 < NCHT)
        def _():
            wait_idx(c + 1, 1 - s)
            mk_tabs(c + 1, 1 - s)
            issue_src(c + 1, 1 - s)

        @pl.when(c >= 2)
        def _():
            wait_vout(c - 2, s)

        _p1_gate(s, bufs, w2v)

        @pl.when(c + 1 < NCHT)
        def _():
            issue_dst(1 - s)

        _p1_msg(s, bufs, acc)
        issue_vout(c, s)

        @pl.when(c + 2 < NCHT)
        def _():
            issue_idx(c + 2, s)

    issue_idx(0, 0)
    wait_idx(0, 0)
    mk_tabs(0, 0)
    issue_src(0, 0)
    issue_dst(0)
    issue_idx(1, 1)

    def pair(c2, carry):
        handler(2 * c2, 0)
        handler(2 * c2 + 1, 1)
        return carry

    lax.fori_loop(0, NCHT // 2, pair, 0)
    wait_vout(NCHT - 2, (NCHT - 2) % 2)
    wait_vout(NCHT - 1, (NCHT - 1) % 2)


def _p1_side(sid, hbm, init_tab, out_tab, acc, bufs, sems, w2buf, w2s):
    _rows_io(sid, init_tab, acc)
    pltpu.sync_copy(w2s, w2buf)
    plsc.subcore_barrier()
    w2a = [w2buf[0, pl.ds(16 * f, 16)] for f in range(8)]
    w2b = [w2buf[8, pl.ds(16 * f, 16)] for f in range(8)]
    _p1_section(sid, hbm, bufs, sems, w2a, w2b, acc)
    plsc.subcore_barrier()
    _rows_io(sid, acc, out_tab)


def _p1_body(u_raw, i_raw, lg_ui, lg_iu,
             tu, ti, su_a, si_a, w2s, sW, eW,
             ue1, ie1, v_ui, v_iu,
             iu0b, iu1b, ir0b, ir1b, it0b, it1b, is0b, is1b, lg0b, lg1b,
             dstb, src0b, src1b, su0b, su1b, si0b, si1b,
             msg, vo0b, vo1b,
             w2buf, acc, sA0, sA1, sB0, sB1, sD, sV0, sV1):
    cid = lax.axis_index("c")
    sid = lax.axis_index("s")
    bufs = ((iu0b, iu1b), (ir0b, ir1b), (it0b, it1b), (is0b, is1b),
            (lg0b, lg1b), dstb,
            (src0b, src1b), (su0b, su1b), (si0b, si1b), msg,
            (vo0b, vo1b))
    sems = ((sA0, sA1), (sB0, sB1), sD, (sV0, sV1))

    @pl.when(cid == 0)
    def _():
        hbm = (u_raw, i_raw, lg_ui, tu, ti, su_a, si_a, v_ui)
        _p1_side(sid, hbm, sW, ue1, acc, bufs, sems, w2buf, w2s)

    @pl.when(cid == 1)
    def _():
        hbm = (i_raw, u_raw, lg_iu, ti, tu, si_a, su_a, v_iu)
        _p1_side(sid, hbm, eW, ie1, acc, bufs, sems, w2buf, w2s)


def _pass1(u_raw, i_raw, lg_ui, lg_iu,
           tu, ti, su_a, si_a, w2s, sW, eW):
    return pl.kernel(
        _p1_body,
        out_type=[_f32((SN, DIM)), _f32((EN, DIM)),
                  _f32((2 * E,)), _f32((2 * E,))],
        mesh=_mesh,
        scratch_types=[
            pltpu.VMEM((K1,), jnp.int32),
            pltpu.VMEM((K1,), jnp.int32),
            pltpu.VMEM((K1,), jnp.int32),
            pltpu.VMEM((K1,), jnp.int32),
            pltpu.VMEM((K1,), jnp.int32),
            pltpu.VMEM((K1,), jnp.int32),
            pltpu.VMEM((K1,), jnp.int32),
            pltpu.VMEM((K1,), jnp.int32),
            pltpu.VMEM((K1,), jnp.float32),
            pltpu.VMEM((K1,), jnp.float32),
            pltpu.VMEM((K1, 2 * DW), jnp.int32),
            pltpu.VMEM((K1, 2 * DW), jnp.int32),
            pltpu.VMEM((K1, 2 * DW), jnp.int32),
            pltpu.VMEM((K1,), jnp.float32),
            pltpu.VMEM((K1,), jnp.float32),
            pltpu.VMEM((K1,), jnp.float32),
            pltpu.VMEM((K1,), jnp.float32),
            pltpu.VMEM((K1, DIM), jnp.float32),
            pltpu.VMEM((K1,), jnp.float32),
            pltpu.VMEM((K1,), jnp.float32),
            pltpu.VMEM((16, DIM), jnp.float32),
            pltpu.VMEM_SHARED((SN, DIM), jnp.float32),
            pltpu.SemaphoreType.DMA,
            pltpu.SemaphoreType.DMA,
            pltpu.SemaphoreType.DMA,
            pltpu.SemaphoreType.DMA,
            pltpu.SemaphoreType.DMA,
            pltpu.SemaphoreType.DMA,
            pltpu.SemaphoreType.DMA,
        ],
    )(u_raw, i_raw, lg_ui, lg_iu, tu, ti, su_a, si_a,
      w2s, sW, eW)


def _p2_compute(s, bufs, acc):
    (ius, iis, vbs, srcs, msg) = bufs
    iu, vb, src = ius[s], vbs[s], srcs[s]

    def group(g, carry):
        e0 = g * 16
        v16 = vb[pl.ds(e0, 16)]
        for j in range(16):
            e = e0 + j
            vs = _takev(v16, jnp.full((16,), j, jnp.int32))
            for w in range(8):
                msg[e, pl.ds(16 * w, 16)] = vs * src[e, pl.ds(16 * w, 16)]
        return carry

    lax.fori_loop(0, K2 // 16, group, 0)
    pltpu.sync_copy(msg, acc.at[ius[s]], add=True)


def _p2_section(sid, hbm, bufs, sems, acc):
    (d_h, s_h, v_h, gatT) = hbm
    (ius, iis, vbs, srcs, msg) = bufs
    (sA, sB) = sems

    def issue_idx(c, s):
        base = sid * EPT2 + c * K2
        pltpu.async_copy(d_h.at[pl.ds(base, K2)], ius[s], sA[s])
        pltpu.async_copy(s_h.at[pl.ds(base, K2)], iis[s], sA[s])
        pltpu.async_copy(v_h.at[pl.ds(base, K2)], vbs[s], sA[s])

    def wait_idx(c, s):
        base = sid * EPT2 + c * K2
        pltpu.make_async_copy(d_h.at[pl.ds(base, K2)], ius[s], sA[s]).wait()
        pltpu.make_async_copy(s_h.at[pl.ds(base, K2)], iis[s], sA[s]).wait()
        pltpu.make_async_copy(v_h.at[pl.ds(base, K2)], vbs[s], sA[s]).wait()

    def issue_rows(c, s):
        pltpu.async_copy(gatT.at[iis[s]], srcs[s], sB[s])

    def wait_rows(c, s):
        pltpu.make_async_copy(gatT.at[iis[s]], srcs[s], sB[s]).wait()

    def handler(c, s):
        wait_rows(c, s)

        @pl.when(c + 1 < NCHT2)
        def _():
            wait_idx(c + 1, 1 - s)
            issue_rows(c + 1, 1 - s)

        _p2_compute(s, bufs, acc)

        @pl.when(c + 2 < NCHT2)
        def _():
            issue_idx(c + 2, s)

    issue_idx(0, 0)
    wait_idx(0, 0)
    issue_rows(0, 0)
    issue_idx(1, 1)

    def pair(c2, carry):
        handler(2 * c2, 0)
        handler(2 * c2 + 1, 1)
        return carry

    lax.fori_loop(0, NCHT2 // 2, pair, 0)


def _p2_side(sid, hbm, init_tab, sel_idx, sel_out, acc, bufs, sems,
             selbufs, extra=None):
    (selidx, selrows, drows, sS) = selbufs
    _rows_io(sid, init_tab, acc)
    plsc.subcore_barrier()
    _p2_section(sid, hbm, bufs, sems, acc)
    plsc.subcore_barrier()
    h0 = sid * SEL_PT
    for cc in range(SEL_PT // SELC):
        hb = h0 + cc * SELC
        pltpu.sync_copy(sel_idx.at[pl.ds(hb, SELC)], selidx)
        pltpu.async_copy(acc.at[selidx], selrows, sS).wait()
        pltpu.sync_copy(selrows, sel_out.at[pl.ds(hb, SELC)])
        if extra is not None:
            disc_t, disc_e = extra
            pltpu.async_copy(disc_t.at[selidx], drows, sS).wait()
            pltpu.sync_copy(drows, disc_e.at[pl.ds(hb, SELC)])


def _p2_body(u_raw, i_raw, v_ui, v_iu,
             ue1, ie1, stu_id, exer_id, disc_t,
             stu_e, ex_e, disc_e,
             iu0b, iu1b, ii0b, ii1b, vb0, vb1, src0b, src1b, msg,
             selidx, selrows, drows,
             acc, sA0, sA1, sB0, sB1, sS):
    cid = lax.axis_index("c")
    sid = lax.axis_index("s")
    bufs = ((iu0b, iu1b), (ii0b, ii1b), (vb0, vb1), (src0b, src1b), msg)
    sems = ((sA0, sA1), (sB0, sB1))
    selbufs = (selidx, selrows, drows, sS)

    @pl.when(cid == 0)
    def _():
        hbm = (u_raw, i_raw, v_ui, ie1)
        _p2_side(sid, hbm, ue1, stu_id, stu_e, acc, bufs, sems, selbufs)

    @pl.when(cid == 1)
    def _():
        hbm = (i_raw, u_raw, v_iu, ue1)
        _p2_side(sid, hbm, ie1, exer_id, ex_e, acc, bufs, sems, selbufs,
                 extra=(disc_t, disc_e))


def _pass2(u_raw, i_raw, v_ui, v_iu, ue1, ie1,
           stu_id, exer_id, disc_t):
    return pl.kernel(
        _p2_body,
        out_type=[_f32((NB, DIM)), _f32((NB, DIM)), _f32((NB,))],
        mesh=_mesh,
        scratch_types=[
            pltpu.VMEM((K2,), jnp.int32),
            pltpu.VMEM((K2,), jnp.int32),
            pltpu.VMEM((K2,), jnp.int32),
            pltpu.VMEM((K2,), jnp.int32),
            pltpu.VMEM((K2,), jnp.float32),
            pltpu.VMEM((K2,), jnp.float32),
            pltpu.VMEM((K2, DIM), jnp.float32),
            pltpu.VMEM((K2, DIM), jnp.float32),
            pltpu.VMEM((K2, DIM), jnp.float32),
            pltpu.VMEM((SELC,), jnp.int32),
            pltpu.VMEM((SELC, DIM), jnp.float32),
            pltpu.VMEM((SELC,), jnp.float32),
            pltpu.VMEM_SHARED((SN, DIM), jnp.float32),
            pltpu.SemaphoreType.DMA,
            pltpu.SemaphoreType.DMA,
            pltpu.SemaphoreType.DMA,
            pltpu.SemaphoreType.DMA,
            pltpu.SemaphoreType.DMA,
        ],
    )(u_raw, i_raw, v_ui, v_iu, ue1, ie1,
      stu_id, exer_id, disc_t)


def _dotT(x, w):
    return lax.dot_general(x, w, (((1,), (1,)), ((), ())),
                           preferred_element_type=jnp.float32)


def _prep_tables_body(sw, ew, w11, b11, w01, b01,
                      s1u, s1i, s0u, s0i):
    swv = sw[...]
    ewv = ew[...]
    swb = swv.astype(jnp.bfloat16)
    ewb = ewv.astype(jnp.bfloat16)
    for wref, bref, su_, si_ in (
            (w11, b11, s1u, s1i), (w01, b01, s0u, s0i)):
        w = wref[...]
        a = (_dotT(swv, w[:, :DIM]) + bref[...]).astype(jnp.bfloat16)
        b = (_dotT(ewv, w[:, DIM:])).astype(jnp.bfloat16)
        su_[...] = jnp.concatenate([a, swb], axis=1)
        si_[...] = jnp.concatenate([b, ewb], axis=1)


def _prep_tables(sW, eW, l1_W1, l1_b1, l0_W1, l0_b1):
    R = 1000
    grid = SN // R
    blk = pl.BlockSpec((R, DIM), lambda i: (i, 0))
    blk2 = pl.BlockSpec((R, 2 * DIM), lambda i: (i, 0))
    full = lambda s: pl.BlockSpec(s, lambda i: tuple(0 for _ in s))
    bf = lambda s: jax.ShapeDtypeStruct(s, jnp.bfloat16)
    return pl.pallas_call(
        _prep_tables_body,
        grid=(grid,),
        in_specs=[blk, blk, full((DIM, 2 * DIM)), full((1, DIM)),
                  full((DIM, 2 * DIM)), full((1, DIM))],
        out_specs=[blk2, blk2, blk2, blk2],
        out_shape=[bf((SN, 2 * DIM)), bf((EN, 2 * DIM)),
                   bf((SN, 2 * DIM)), bf((EN, 2 * DIM))],
    )(sW, eW, l1_W1, l1_b1.reshape(1, DIM), l0_W1, l0_b1.reshape(1, DIM))


def _prep_small_body(degp, edisc, su4, disc):
    d = degp[...]
    su4[...] = lax.rsqrt(d[0] + d[1] + 1.0)
    disc[...] = jax.nn.sigmoid(edisc[...]) * 10.0


def _prep_small(degp, e_disc_W):
    return pl.pallas_call(
        _prep_small_body,
        out_shape=[_f32((4, SN)), _f32((1, EN))],
    )(degp, e_disc_W)


def _prep_edge_body(e1, e2, e3, e4, b21, b20, o1, o2, o3, o4):
    def lg(x, b2):
        xc = jnp.clip(x, 1e-6, 1.0 - 1e-6)
        return 5.0 * (jnp.log(xc) - jnp.log1p(-xc) + b2)
    b1v = b21[0, 0]
    b0v = b20[0, 0]
    o1[...] = lg(e1[...], b1v)
    o2[...] = lg(e2[...], b1v)
    o3[...] = lg(e3[...], b0v)
    o4[...] = lg(e4[...], b0v)


def _prep_edge(eps_ui_1, eps_iu_1, eps_ui_0, eps_iu_0, l1_b2, l0_b2):
    W = 128
    rs = lambda x: x.reshape(E // W, W)
    outs = pl.pallas_call(
        _prep_edge_body,
        out_shape=[_f32((E // W, W))] * 4,
    )(rs(eps_ui_1), rs(eps_iu_1), rs(eps_ui_0), rs(eps_iu_0),
      l1_b2.reshape(1, 1), l0_b2.reshape(1, 1))
    return [o.reshape(E) for o in outs]


def _head_body(stu, ex, disc, knr, knw, p1w, p1b, p2w, p2b, p3w, p3b, out):
    stat = jax.nn.sigmoid(_dotT(stu[...], knw[...]))
    kdiff = jax.nn.sigmoid(_dotT(ex[...], knw[...]))
    x = disc[...] * (stat - kdiff) * knr[...]
    h1 = jax.nn.sigmoid(_dotT(x, jnp.abs(p1w[...])) + p1b[...])
    h2 = jax.nn.sigmoid(_dotT(h1, jnp.abs(p2w[...])) + p2b[...])
    w3 = jnp.concatenate([jnp.abs(p3w[...]), jnp.zeros((127, DIM), jnp.float32)],
                         axis=0)
    r = _dotT(h2, w3)
    out[...] = jax.nn.sigmoid(r[:, 0:1] + p3b[0, 0])


def _head(stu_e, ex_e, disc_e, kn_r, knowledge_W,
          pn1_W, pn1_b, pn2_W, pn2_b, pn3_W, pn3_b):
    R = 512
    grid = NB // R
    blk = pl.BlockSpec((R, DIM), lambda i: (i, 0))
    blk1 = pl.BlockSpec((R, 1), lambda i: (i, 0))
    full = lambda s: pl.BlockSpec(s, lambda i: tuple(0 for _ in s))
    out = pl.pallas_call(
        _head_body,
        grid=(grid,),
        in_specs=[blk, blk, blk1, blk,
                  full((DIM, DIM)),
                  full((256, DIM)), full((1, 256)),
                  full((DIM, 256)), full((1, DIM)),
                  full((1, DIM)), full((1, 1))],
        out_specs=blk1,
        out_shape=_f32((NB, 1)),
    )(stu_e, ex_e, disc_e.reshape(NB, 1), kn_r, knowledge_W,
      pn1_W, pn1_b.reshape(1, 256), pn2_W, pn2_b.reshape(1, DIM),
      pn3_W, pn3_b.reshape(1, 1))
    return out.reshape(NB)


def _packw(t):
    n, f = t.shape
    arr = t.reshape(n, f // 32, 2, 16).transpose(0, 1, 3, 2)
    return lax.bitcast_convert_type(arr, jnp.int32).reshape(n, f // 2)


def kernel(stu_id, exer_id, kn_r, edge_index_1, edge_index_0,
           eps_ui_1, eps_iu_1, eps_ui_0, eps_iu_0,
           student_W, exercise_W, knowledge_W, e_disc_W,
           l1_W1, l1_b1, l1_W2, l1_b2,
           l0_W1, l0_b1, l0_W2, l0_b2,
           pn1_W, pn1_b, pn2_W, pn2_b, pn3_W, pn3_b):
    u1 = edge_index_1[0]
    i1 = edge_index_1[1]
    u0 = edge_index_0[0]
    i0 = edge_index_0[1]

    degp = _pass0(u1, i1, u0, i0)
    su4, disc = _prep_small(degp.reshape(2, 4, SN), e_disc_W.reshape(1, EN))
    su1, si1, su0, si0 = su4[0], su4[1], su4[2], su4[3]
    disc_t = disc.reshape(EN)

    s1u, s1i, s0u, s0i = _prep_tables(
        student_W, exercise_W, l1_W1, l1_b1, l0_W1, l0_b1)
    tu = jnp.concatenate([_packw(s1u), _packw(s0u)], axis=0)
    ti = jnp.concatenate([_packw(s1i), _packw(s0i)], axis=0)
    su_a = jnp.concatenate([su1, su0])
    si_a = jnp.concatenate([si1, si0])
    lgs = _prep_edge(eps_ui_1, eps_iu_1, eps_ui_0, eps_iu_0, l1_b2, l0_b2)

    tile = lambda x: x.reshape(NT, EPT)
    cat2 = lambda a, b: jnp.concatenate([tile(a), tile(b)], axis=1).reshape(-1)
    u_raw = cat2(u1, u0)
    i_raw = cat2(i1, i0)
    lg_ui = cat2(lgs[0], lgs[2])
    lg_iu = cat2(lgs[1], lgs[3])

    w2s = jnp.zeros((16, DIM), jnp.float32)
    w2s = w2s.at[0].set(l1_W2[0] * 5.0).at[8].set(l0_W2[0] * 5.0)

    ue1, ie1, v_ui, v_iu = _pass1(
        u_raw, i_raw, lg_ui, lg_iu,
        tu, ti, su_a, si_a, w2s, student_W, exercise_W)

    stu_e, ex_e, disc_e = _pass2(
        u_raw, i_raw, v_ui, v_iu, ue1, ie1, stu_id, exer_id, disc_t)

    return _head(stu_e, ex_e, disc_e, kn_r, knowledge_W,
                 pn1_W, pn1_b, pn2_W, pn2_b, pn3_W, pn3_b)


# speedup vs baseline: 7.4854x; 1.0013x over previous
import jax
import jax.numpy as jnp
from jax import lax
from jax.experimental import pallas as pl
from jax.experimental.pallas import tpu as pltpu
from jax.experimental.pallas import tpu_sc as plsc

SN = 10000
EN = 10000
DIM = 128
E = 320000
NB = 4096
CONTROL = 0.3
NT = 16
EPT = E // NT
K1 = 80
K2 = 80
K0 = 2000
NCH0 = (E // 8) // K0
SEL_PT = NB // NT
SELC = 64
DW = DIM // 2
ROWS_SPLIT = [(t * 632, 632) for t in range(15)] + [(9480, 520)]

def _mesh_sc():
    return plsc.VectorSubcoreMesh(core_axis_name="c", subcore_axis_name="s")


def _f32(shape):
    return jax.ShapeDtypeStruct(shape, jnp.float32)


def _takev(v, idx):
    return v.at[idx].get(mode="promise_in_bounds")


def _lo(x):
    return lax.bitcast_convert_type(lax.shift_left(x, 16), jnp.float32)


def _hi(x):
    return lax.bitcast_convert_type(jnp.bitwise_and(x, jnp.int32(-65536)),
                                    jnp.float32)


def _rows_io(sid, src, dst):
    for t, (b, s) in enumerate(ROWS_SPLIT):
        @pl.when(sid == t)
        def _(b=b, s=s):
            pltpu.sync_copy(src.at[pl.ds(b, s)], dst.at[pl.ds(b, s)])


def _hsum16(accs):
    lane = lax.iota(jnp.int32, 16)
    cur = accs
    for fold, rotk in ((8, 8), (4, 12), (2, 14), (1, 15)):
        fidx = jnp.bitwise_and(lane + fold, 15)
        ridx = jnp.bitwise_and(lane + rotk, 15)
        m = jnp.bitwise_and(lane, 2 * fold - 1) < fold
        nxt = []
        for a, b in zip(cur[0::2], cur[1::2]):
            fa = a + _takev(a, fidx)
            fb = b + _takev(b, fidx)
            nxt.append(jnp.where(m, fa, _takev(fb, ridx)))
        cur = nxt
    perm = (jnp.bitwise_and(lane, 1) * 8 + jnp.bitwise_and(lane, 2) * 2
            + lax.shift_right_logical(jnp.bitwise_and(lane, 4), 1)
            + lax.shift_right_logical(jnp.bitwise_and(lane, 8), 3))
    return _takev(cur[0], perm)


def _p0_body(u1, i1, u0, i0, ones_h, degp,
             idxb, onesb, zbuf, a0, a1, a2, a3):
    cid = lax.axis_index("c")
    sid = lax.axis_index("s")
    accs = (a0, a1, a2, a3)
    arrays = (u1, i1, u0, i0)

    @pl.when(sid == 0)
    def _():
        def zfill(i, c):
            zbuf[pl.ds(i * 16, 16)] = jnp.zeros((16,), jnp.float32)
            return c
        lax.fori_loop(0, SN // 16, zfill, 0)
        for k in range(4):
            pltpu.sync_copy(zbuf, accs[k])

    pltpu.sync_copy(ones_h, onesb)
    plsc.subcore_barrier()

    a = sid // 4
    q = jnp.remainder(sid, 4)
    wrk = cid * 4 + q

    def chunk(c, carry):
        base = wrk * (E // 8) + c * K0
        for k in range(4):
            @pl.when(a == k)
            def _(k=k):
                pltpu.sync_copy(arrays[k].at[pl.ds(base, K0)], idxb)
                pltpu.sync_copy(onesb, accs[k].at[idxb], add=True)
        return carry

    lax.fori_loop(0, NCH0, chunk, 0)
    plsc.subcore_barrier()
    for k in range(4):
        @pl.when(sid == k)
        def _(k=k):
            pltpu.sync_copy(accs[k], zbuf)
            pltpu.sync_copy(zbuf, degp.at[pl.ds((cid * 4 + k) * SN, SN)])


def _pass0(u1, i1, u0, i0):
    ones_h = jnp.ones((K0,), jnp.float32)
    return pl.kernel(
        _p0_body,
        out_type=_f32((8 * SN,)),
        mesh=_mesh_sc(),
        scratch_types=[
            pltpu.VMEM((K0,), jnp.int32),
            pltpu.VMEM((K0,), jnp.float32),
            pltpu.VMEM((SN,), jnp.float32),
            pltpu.VMEM_SHARED((SN,), jnp.float32),
            pltpu.VMEM_SHARED((SN,), jnp.float32),
            pltpu.VMEM_SHARED((SN,), jnp.float32),
            pltpu.VMEM_SHARED((SN,), jnp.float32),
        ],
    )(u1, i1, u0, i0, ones_h)


EPT2 = 2 * EPT
NCHT = EPT2 // K1
NCHT2 = EPT2 // K2


def _p1_gate(s, bufs, w2v):
    (ius, irs, its, iss, lgs_, dstb, srcs, sus, sis, msg, vouts) = bufs
    lg, src, vout = lgs_[s], srcs[s], vouts[s]
    su_e, si_e = sus[s], sis[s]

    def group(g, carry):
        e0 = g * 16
        accs = []
        for j in range(16):
            e = e0 + j
            acc16 = None
            for w in range(4):
                dI = dstb[e, pl.ds(16 * w, 16)]
                sI = src[e, pl.ds(16 * w, 16)]
                hl = jnp.maximum(_lo(dI) + _lo(sI), 0.0)
                hh = jnp.maximum(_hi(dI) + _hi(sI), 0.0)
                t = hl * w2v[2 * w] + hh * w2v[2 * w + 1]
                acc16 = t if acc16 is None else acc16 + t
            accs.append(acc16)
        logit = _hsum16(accs)
        gv = logit + lg[pl.ds(e0, 16)]
        wgt = CONTROL / (1.0 + jnp.exp(-gv)) + (1.0 - CONTROL)
        v = su_e[pl.ds(e0, 16)] * si_e[pl.ds(e0, 16)] * wgt
        vout[pl.ds(e0, 16)] = v
        return carry

    lax.fori_loop(0, K1 // 16, group, 0)


def _p1_msg(s, bufs, acc):
    (ius, irs, its, iss, lgs_, dstb, srcs, sus, sis, msg, vouts) = bufs
    src, vout = srcs[s], vouts[s]

    def group(g, carry):
        e0 = g * 16
        v16 = vout[pl.ds(e0, 16)]
        for j in range(16):
            e = e0 + j
            vs = _takev(v16, jnp.full((16,), j, jnp.int32))
            for w in range(4):
                mI = src[e, pl.ds(DW + 16 * w, 16)]
                msg[e, pl.ds(32 * w, 16)] = vs * _lo(mI)
                msg[e, pl.ds(32 * w + 16, 16)] = vs * _hi(mI)
        return carry

    lax.fori_loop(0, K1 // 16, group, 0)
    pltpu.sync_copy(msg, acc.at[ius[s]], add=True)


def _p1_section(sid, hbm, bufs, sems, w2a, w2b, acc):
    (dr_h, sr_h, lg_h, dstT, srcT, su_t, si_t, v_h) = hbm
    (ius, irs, its, iss, lgs_, dstb, srcs, sus, sis, msg, vouts) = bufs
    (sA, sB, sD, sV) = sems

    def issue_idx(c, s):
        base = sid * EPT2 + c * K1
        pltpu.async_copy(dr_h.at[pl.ds(base, K1)], ius[s], sA[s])
        pltpu.async_copy(sr_h.at[pl.ds(base, K1)], irs[s], sA[s])
        pltpu.async_copy(lg_h.at[pl.ds(base, K1)], lgs_[s], sA[s])

    def wait_idx(c, s):
        base = sid * EPT2 + c * K1
        pltpu.make_async_copy(dr_h.at[pl.ds(base, K1)], ius[s], sA[s]).wait()
        pltpu.make_async_copy(sr_h.at[pl.ds(base, K1)], irs[s], sA[s]).wait()
        pltpu.make_async_copy(lg_h.at[pl.ds(base, K1)], lgs_[s], sA[s]).wait()

    def mk_tabs(c, s):
        offv = jnp.where(c < NCHT // 2, jnp.zeros((16,), jnp.int32),
                         jnp.full((16,), SN, jnp.int32))
        for g in range(K1 // 16):
            e0 = g * 16
            its[s][pl.ds(e0, 16)] = ius[s][pl.ds(e0, 16)] + offv
            iss[s][pl.ds(e0, 16)] = irs[s][pl.ds(e0, 16)] + offv

    def issue_src(c, s):
        pltpu.async_copy(srcT.at[iss[s]], srcs[s], sB[s])
        pltpu.async_copy(su_t.at[its[s]], sus[s], sB[s])
        pltpu.async_copy(si_t.at[iss[s]], sis[s], sB[s])

    def wait_src(c, s):
        pltpu.make_async_copy(srcT.at[iss[s]], srcs[s], sB[s]).wait()
        pltpu.make_async_copy(su_t.at[its[s]], sus[s], sB[s]).wait()
        pltpu.make_async_copy(si_t.at[iss[s]], sis[s], sB[s]).wait()

    def issue_dst(s):
        pltpu.async_copy(dstT.at[its[s]], dstb, sD)

    def wait_dst(s):
        pltpu.make_async_copy(dstT.at[its[s]], dstb, sD).wait()

    def issue_vout(c, s):
        base = sid * EPT2 + c * K1
        pltpu.async_copy(vouts[s], v_h.at[pl.ds(base, K1)], sV[s])

    def wait_vout(c, s):
        base = sid * EPT2 + c * K1
        pltpu.make_async_copy(vouts[s], v_h.at[pl.ds(base, K1)], sV[s]).wait()

    def handler(c, s):
        w2v = [jnp.where(c < NCHT // 2, w2a[f], w2b[f]) for f in range(8)]
        wait_src(c, s)
        wait_dst(s)

        @pl.when(c + 1 < NCHT)
        def _():
            wait_idx(c + 1, 1 - s)
            mk_tabs(c + 1, 1 - s)
            issue_src(c + 1, 1 - s)

        @pl.when(c >= 2)
        def _():
            wait_vout(c - 2, s)

        _p1_gate(s, bufs, w2v)

        @pl.when(c + 1 < NCHT)
        def _():
            issue_dst(1 - s)

        _p1_msg(s, bufs, acc)
        issue_vout(c, s)

        @pl.when(c + 2 < NCHT)
        def _():
            issue_idx(c + 2, s)

    issue_idx(0, 0)
    wait_idx(0, 0)
    mk_tabs(0, 0)
    issue_src(0, 0)
    issue_dst(0)
    issue_idx(1, 1)

    def pair(c2, carry):
        handler(2 * c2, 0)
        handler(2 * c2 + 1, 1)
        return carry

    lax.fori_loop(0, NCHT // 2, pair, 0)
    wait_vout(NCHT - 2, (NCHT - 2) % 2)
    wait_vout(NCHT - 1, (NCHT - 1) % 2)


def _p1_side(sid, hbm, init_tab, out_tab, acc, bufs, sems, w2buf, w2s):
    _rows_io(sid, init_tab, acc)
    pltpu.sync_copy(w2s, w2buf)
    plsc.subcore_barrier()
    w2a = [w2buf[0, pl.ds(16 * f, 16)] for f in range(8)]
    w2b = [w2buf[8, pl.ds(16 * f, 16)] for f in range(8)]
    _p1_section(sid, hbm, bufs, sems, w2a, w2b, acc)
    plsc.subcore_barrier()
    _rows_io(sid, acc, out_tab)


def _p1_body(u_raw, i_raw, lg_ui, lg_iu,
             tu, ti, su_a, si_a, w2s, sW, eW,
             ue1, ie1, v_ui, v_iu,
             iu0b, iu1b, ir0b, ir1b, it0b, it1b, is0b, is1b, lg0b, lg1b,
             dstb, src0b, src1b, su0b, su1b, si0b, si1b,
             msg, vo0b, vo1b,
             w2buf, acc, sA0, sA1, sB0, sB1, sD, sV0, sV1):
    cid = lax.axis_index("c")
    sid = lax.axis_index("s")
    bufs = ((iu0b, iu1b), (ir0b, ir1b), (it0b, it1b), (is0b, is1b),
            (lg0b, lg1b), dstb,
            (src0b, src1b), (su0b, su1b), (si0b, si1b), msg,
            (vo0b, vo1b))
    sems = ((sA0, sA1), (sB0, sB1), sD, (sV0, sV1))

    @pl.when(cid == 0)
    def _():
        hbm = (u_raw, i_raw, lg_ui, tu, ti, su_a, si_a, v_ui)
        _p1_side(sid, hbm, sW, ue1, acc, bufs, sems, w2buf, w2s)

    @pl.when(cid == 1)
    def _():
        hbm = (i_raw, u_raw, lg_iu, ti, tu, si_a, su_a, v_iu)
        _p1_side(sid, hbm, eW, ie1, acc, bufs, sems, w2buf, w2s)


def _pass1(u_raw, i_raw, lg_ui, lg_iu,
           tu, ti, su_a, si_a, w2s, sW, eW):
    return pl.kernel(
        _p1_body,
        out_type=[_f32((SN, DIM)), _f32((EN, DIM)),
                  _f32((2 * E,)), _f32((2 * E,))],
        mesh=_mesh_sc(),
        scratch_types=[
            pltpu.VMEM((K1,), jnp.int32),
            pltpu.VMEM((K1,), jnp.int32),
            pltpu.VMEM((K1,), jnp.int32),
            pltpu.VMEM((K1,), jnp.int32),
            pltpu.VMEM((K1,), jnp.int32),
            pltpu.VMEM((K1,), jnp.int32),
            pltpu.VMEM((K1,), jnp.int32),
            pltpu.VMEM((K1,), jnp.int32),
            pltpu.VMEM((K1,), jnp.float32),
            pltpu.VMEM((K1,), jnp.float32),
            pltpu.VMEM((K1, 2 * DW), jnp.int32),
            pltpu.VMEM((K1, 2 * DW), jnp.int32),
            pltpu.VMEM((K1, 2 * DW), jnp.int32),
            pltpu.VMEM((K1,), jnp.float32),
            pltpu.VMEM((K1,), jnp.float32),
            pltpu.VMEM((K1,), jnp.float32),
            pltpu.VMEM((K1,), jnp.float32),
            pltpu.VMEM((K1, DIM), jnp.float32),
            pltpu.VMEM((K1,), jnp.float32),
            pltpu.VMEM((K1,), jnp.float32),
            pltpu.VMEM((16, DIM), jnp.float32),
            pltpu.VMEM_SHARED((SN, DIM), jnp.float32),
            pltpu.SemaphoreType.DMA,
            pltpu.SemaphoreType.DMA,
            pltpu.SemaphoreType.DMA,
            pltpu.SemaphoreType.DMA,
            pltpu.SemaphoreType.DMA,
            pltpu.SemaphoreType.DMA,
            pltpu.SemaphoreType.DMA,
        ],
    )(u_raw, i_raw, lg_ui, lg_iu, tu, ti, su_a, si_a,
      w2s, sW, eW)


def _p2_compute(s, bufs, acc):
    (ius, iis, vbs, srcs, msg) = bufs
    iu, vb, src = ius[s], vbs[s], srcs[s]

    def group(g, carry):
        e0 = g * 16
        v16 = vb[pl.ds(e0, 16)]
        for j in range(16):
            e = e0 + j
            vs = _takev(v16, jnp.full((16,), j, jnp.int32))
            for w in range(8):
                msg[e, pl.ds(16 * w, 16)] = vs * src[e, pl.ds(16 * w, 16)]
        return carry

    lax.fori_loop(0, K2 // 16, group, 0)
    pltpu.sync_copy(msg, acc.at[ius[s]], add=True)


def _p2_section(sid, hbm, bufs, sems, acc):
    (d_h, s_h, v_h, gatT) = hbm
    (ius, iis, vbs, srcs, msg) = bufs
    (sA, sB) = sems

    def issue_idx(c, s):
        base = sid * EPT2 + c * K2
        pltpu.async_copy(d_h.at[pl.ds(base, K2)], ius[s], sA[s])
        pltpu.async_copy(s_h.at[pl.ds(base, K2)], iis[s], sA[s])
        pltpu.async_copy(v_h.at[pl.ds(base, K2)], vbs[s], sA[s])

    def wait_idx(c, s):
        base = sid * EPT2 + c * K2
        pltpu.make_async_copy(d_h.at[pl.ds(base, K2)], ius[s], sA[s]).wait()
        pltpu.make_async_copy(s_h.at[pl.ds(base, K2)], iis[s], sA[s]).wait()
        pltpu.make_async_copy(v_h.at[pl.ds(base, K2)], vbs[s], sA[s]).wait()

    def issue_rows(c, s):
        pltpu.async_copy(gatT.at[iis[s]], srcs[s], sB[s])

    def wait_rows(c, s):
        pltpu.make_async_copy(gatT.at[iis[s]], srcs[s], sB[s]).wait()

    def handler(c, s):
        wait_rows(c, s)

        @pl.when(c + 1 < NCHT2)
        def _():
            wait_idx(c + 1, 1 - s)
            issue_rows(c + 1, 1 - s)

        _p2_compute(s, bufs, acc)

        @pl.when(c + 2 < NCHT2)
        def _():
            issue_idx(c + 2, s)

    issue_idx(0, 0)
    wait_idx(0, 0)
    issue_rows(0, 0)
    issue_idx(1, 1)

    def pair(c2, carry):
        handler(2 * c2, 0)
        handler(2 * c2 + 1, 1)
        return carry

    lax.fori_loop(0, NCHT2 // 2, pair, 0)


def _p2_side(sid, hbm, init_tab, sel_idx, sel_out, acc, bufs, sems,
             selbufs, extra=None):
    (selidx, selrows, drows, sS) = selbufs
    _rows_io(sid, init_tab, acc)
    plsc.subcore_barrier()
    _p2_section(sid, hbm, bufs, sems, acc)
    plsc.subcore_barrier()
    h0 = sid * SEL_PT
    for cc in range(SEL_PT // SELC):
        hb = h0 + cc * SELC
        pltpu.sync_copy(sel_idx.at[pl.ds(hb, SELC)], selidx)
        pltpu.async_copy(acc.at[selidx], selrows, sS).wait()
        pltpu.sync_copy(selrows, sel_out.at[pl.ds(hb, SELC)])
        if extra is not None:
            disc_t, disc_e = extra
            pltpu.async_copy(disc_t.at[selidx], drows, sS).wait()
            pltpu.sync_copy(drows, disc_e.at[pl.ds(hb, SELC)])


def _p2_body(u_raw, i_raw, v_ui, v_iu,
             ue1, ie1, stu_id, exer_id, disc_t,
             stu_e, ex_e, disc_e,
             iu0b, iu1b, ii0b, ii1b, vb0, vb1, src0b, src1b, msg,
             selidx, selrows, drows,
             acc, sA0, sA1, sB0, sB1, sS):
    cid = lax.axis_index("c")
    sid = lax.axis_index("s")
    bufs = ((iu0b, iu1b), (ii0b, ii1b), (vb0, vb1), (src0b, src1b), msg)
    sems = ((sA0, sA1), (sB0, sB1))
    selbufs = (selidx, selrows, drows, sS)

    @pl.when(cid == 0)
    def _():
        hbm = (u_raw, i_raw, v_ui, ie1)
        _p2_side(sid, hbm, ue1, stu_id, stu_e, acc, bufs, sems, selbufs)

    @pl.when(cid == 1)
    def _():
        hbm = (i_raw, u_raw, v_iu, ue1)
        _p2_side(sid, hbm, ie1, exer_id, ex_e, acc, bufs, sems, selbufs,
                 extra=(disc_t, disc_e))


def _pass2(u_raw, i_raw, v_ui, v_iu, ue1, ie1,
           stu_id, exer_id, disc_t):
    return pl.kernel(
        _p2_body,
        out_type=[_f32((NB, DIM)), _f32((NB, DIM)), _f32((NB,))],
        mesh=_mesh_sc(),
        scratch_types=[
            pltpu.VMEM((K2,), jnp.int32),
            pltpu.VMEM((K2,), jnp.int32),
            pltpu.VMEM((K2,), jnp.int32),
            pltpu.VMEM((K2,), jnp.int32),
            pltpu.VMEM((K2,), jnp.float32),
            pltpu.VMEM((K2,), jnp.float32),
            pltpu.VMEM((K2, DIM), jnp.float32),
            pltpu.VMEM((K2, DIM), jnp.float32),
            pltpu.VMEM((K2, DIM), jnp.float32),
            pltpu.VMEM((SELC,), jnp.int32),
            pltpu.VMEM((SELC, DIM), jnp.float32),
            pltpu.VMEM((SELC,), jnp.float32),
            pltpu.VMEM_SHARED((SN, DIM), jnp.float32),
            pltpu.SemaphoreType.DMA,
            pltpu.SemaphoreType.DMA,
            pltpu.SemaphoreType.DMA,
            pltpu.SemaphoreType.DMA,
            pltpu.SemaphoreType.DMA,
        ],
    )(u_raw, i_raw, v_ui, v_iu, ue1, ie1,
      stu_id, exer_id, disc_t)


def _dotT(x, w):
    return lax.dot_general(x, w, (((1,), (1,)), ((), ())),
                           preferred_element_type=jnp.float32)


def _prep_tables_body(sw, ew, w11, b11, w01, b01,
                      s1u, s1i, s0u, s0i):
    swv = sw[...]
    ewv = ew[...]
    swb = swv.astype(jnp.bfloat16)
    ewb = ewv.astype(jnp.bfloat16)
    for wref, bref, su_, si_ in (
            (w11, b11, s1u, s1i), (w01, b01, s0u, s0i)):
        w = wref[...]
        a = (_dotT(swv, w[:, :DIM]) + bref[...]).astype(jnp.bfloat16)
        b = (_dotT(ewv, w[:, DIM:])).astype(jnp.bfloat16)
        su_[...] = jnp.concatenate([a, swb], axis=1)
        si_[...] = jnp.concatenate([b, ewb], axis=1)


def _prep_tables(sW, eW, l1_W1, l1_b1, l0_W1, l0_b1):
    R = 1000
    grid = SN // R
    blk = pl.BlockSpec((R, DIM), lambda i: (i, 0))
    blk2 = pl.BlockSpec((R, 2 * DIM), lambda i: (i, 0))
    full = lambda s: pl.BlockSpec(s, lambda i: tuple(0 for _ in s))
    bf = lambda s: jax.ShapeDtypeStruct(s, jnp.bfloat16)
    return pl.pallas_call(
        _prep_tables_body,
        grid=(grid,),
        in_specs=[blk, blk, full((DIM, 2 * DIM)), full((1, DIM)),
                  full((DIM, 2 * DIM)), full((1, DIM))],
        out_specs=[blk2, blk2, blk2, blk2],
        out_shape=[bf((SN, 2 * DIM)), bf((EN, 2 * DIM)),
                   bf((SN, 2 * DIM)), bf((EN, 2 * DIM))],
    )(sW, eW, l1_W1, l1_b1.reshape(1, DIM), l0_W1, l0_b1.reshape(1, DIM))


def _prep_small_body(degp, edisc, su4, disc):
    d = degp[...]
    su4[...] = lax.rsqrt(d[0] + d[1] + 1.0)
    disc[...] = jax.nn.sigmoid(edisc[...]) * 10.0


def _prep_small(degp, e_disc_W):
    return pl.pallas_call(
        _prep_small_body,
        out_shape=[_f32((4, SN)), _f32((1, EN))],
    )(degp, e_disc_W)


def _prep_edge_body(e1, e2, e3, e4, b21, b20, o1, o2, o3, o4):
    def lg(x, b2):
        xc = jnp.clip(x, 1e-6, 1.0 - 1e-6)
        return 5.0 * (jnp.log(xc) - jnp.log1p(-xc) + b2)
    b1v = b21[0, 0]
    b0v = b20[0, 0]
    o1[...] = lg(e1[...], b1v)
    o2[...] = lg(e2[...], b1v)
    o3[...] = lg(e3[...], b0v)
    o4[...] = lg(e4[...], b0v)


def _prep_edge(eps_ui_1, eps_iu_1, eps_ui_0, eps_iu_0, l1_b2, l0_b2):
    W = 128
    rs = lambda x: x.reshape(E // W, W)
    outs = pl.pallas_call(
        _prep_edge_body,
        out_shape=[_f32((E // W, W))] * 4,
    )(rs(eps_ui_1), rs(eps_iu_1), rs(eps_ui_0), rs(eps_iu_0),
      l1_b2.reshape(1, 1), l0_b2.reshape(1, 1))
    return [o.reshape(E) for o in outs]


def _head_body(stu, ex, disc, knr, knw, p1w, p1b, p2w, p2b, p3w, p3b, out):
    stat = jax.nn.sigmoid(_dotT(stu[...], knw[...]))
    kdiff = jax.nn.sigmoid(_dotT(ex[...], knw[...]))
    x = disc[...] * (stat - kdiff) * knr[...]
    h1 = jax.nn.sigmoid(_dotT(x, jnp.abs(p1w[...])) + p1b[...])
    h2 = jax.nn.sigmoid(_dotT(h1, jnp.abs(p2w[...])) + p2b[...])
    w3 = jnp.concatenate([jnp.abs(p3w[...]), jnp.zeros((127, DIM), jnp.float32)],
                         axis=0)
    r = _dotT(h2, w3)
    out[...] = jax.nn.sigmoid(r[:, 0:1] + p3b[0, 0])


def _head(stu_e, ex_e, disc_e, kn_r, knowledge_W,
          pn1_W, pn1_b, pn2_W, pn2_b, pn3_W, pn3_b):
    R = 512
    grid = NB // R
    blk = pl.BlockSpec((R, DIM), lambda i: (i, 0))
    blk1 = pl.BlockSpec((R, 1), lambda i: (i, 0))
    full = lambda s: pl.BlockSpec(s, lambda i: tuple(0 for _ in s))
    out = pl.pallas_call(
        _head_body,
        grid=(grid,),
        in_specs=[blk, blk, blk1, blk,
                  full((DIM, DIM)),
                  full((256, DIM)), full((1, 256)),
                  full((DIM, 256)), full((1, DIM)),
                  full((1, DIM)), full((1, 1))],
        out_specs=blk1,
        out_shape=_f32((NB, 1)),
    )(stu_e, ex_e, disc_e.reshape(NB, 1), kn_r, knowledge_W,
      pn1_W, pn1_b.reshape(1, 256), pn2_W, pn2_b.reshape(1, DIM),
      pn3_W, pn3_b.reshape(1, 1))
    return out.reshape(NB)


def _packw(t):
    n, f = t.shape
    arr = t.reshape(n, f // 32, 2, 16).transpose(0, 1, 3, 2)
    return lax.bitcast_convert_type(arr, jnp.int32).reshape(n, f // 2)


def kernel(stu_id, exer_id, kn_r, edge_index_1, edge_index_0,
           eps_ui_1, eps_iu_1, eps_ui_0, eps_iu_0,
           student_W, exercise_W, knowledge_W, e_disc_W,
           l1_W1, l1_b1, l1_W2, l1_b2,
           l0_W1, l0_b1, l0_W2, l0_b2,
           pn1_W, pn1_b, pn2_W, pn2_b, pn3_W, pn3_b):
    u1 = edge_index_1[0]
    i1 = edge_index_1[1]
    u0 = edge_index_0[0]
    i0 = edge_index_0[1]

    degp = _pass0(u1, i1, u0, i0)
    su4, disc = _prep_small(degp.reshape(2, 4, SN), e_disc_W.reshape(1, EN))
    su1, si1, su0, si0 = su4[0], su4[1], su4[2], su4[3]
    disc_t = disc.reshape(EN)

    s1u, s1i, s0u, s0i = _prep_tables(
        student_W, exercise_W, l1_W1, l1_b1, l0_W1, l0_b1)
    tu = jnp.concatenate([_packw(s1u), _packw(s0u)], axis=0)
    ti = jnp.concatenate([_packw(s1i), _packw(s0i)], axis=0)
    su_a = jnp.concatenate([su1, su0])
    si_a = jnp.concatenate([si1, si0])
    lgs = _prep_edge(eps_ui_1, eps_iu_1, eps_ui_0, eps_iu_0, l1_b2, l0_b2)

    tile = lambda x: x.reshape(NT, EPT)
    cat2 = lambda a, b: jnp.concatenate([tile(a), tile(b)], axis=1).reshape(-1)
    u_raw = cat2(u1, u0)
    i_raw = cat2(i1, i0)
    lg_ui = cat2(lgs[0], lgs[2])
    lg_iu = cat2(lgs[1], lgs[3])

    w2s = jnp.zeros((16, DIM), jnp.float32)
    w2s = w2s.at[0].set(l1_W2[0] * 5.0).at[8].set(l0_W2[0] * 5.0)

    ue1, ie1, v_ui, v_iu = _pass1(
        u_raw, i_raw, lg_ui, lg_iu,
        tu, ti, su_a, si_a, w2s, student_W, exercise_W)

    stu_e, ex_e, disc_e = _pass2(
        u_raw, i_raw, v_ui, v_iu, ue1, ie1, stu_id, exer_id, disc_t)

    return _head(stu_e, ex_e, disc_e, kn_r, knowledge_W,
                 pn1_W, pn1_b, pn2_W, pn2_b, pn3_W, pn3_b)
